# Initial kernel scaffold; baseline (speedup 1.0000x reference)
import jax
import jax.numpy as jnp
from jax.experimental import pallas as pl


def kernel(nfeats, edge_index, efeats, Wa1, ba1, We1, be1, Wa2, ba2, We2, be2):
    raise NotImplementedError("write your pallas kernel here")


# speedup vs baseline: 3.8173x; 3.8173x over previous
import functools

import jax
import jax.numpy as jnp
from jax import lax
from jax.experimental import pallas as pl
from jax.experimental.pallas import tpu as pltpu
from jax.experimental.pallas import tpu_sc as plsc

N = 10000
NP = 10240
E = 320000
DN = 128
DE = 16
H = 128

CH = 128
NCHUNK = E // CH
NC = 2
NS = 16
NW = NC * NS
ROWS_PER_TILE = NP // NS


def _zero_2d(ref, nrows, ncols):
  z = jnp.zeros((16,), jnp.float32)

  def body(r, _):
    for k in range(ncols // 16):
      ref[r, pl.ds(k * 16, 16)] = z
    return 0

  lax.fori_loop(0, nrows, body, 0)


def _fill_ones_2d(ref, nrows, ncols):
  o = jnp.ones((16,), jnp.float32)

  def body(r, _):
    for k in range(ncols // 16):
      ref[r, pl.ds(k * 16, 16)] = o
    return 0

  lax.fori_loop(0, nrows, body, 0)


def _relu_add_rows(dst, src, nrows, ncols):
  def body(r, _):
    for k in range(ncols // 16):
      s = pl.ds(k * 16, 16)
      dst[r, s] = jnp.maximum(dst[r, s] + src[r, s], 0.0)
    return 0

  lax.fori_loop(0, nrows, body, 0)


def _sc_pass_a1(u, v, nfeats):
  mesh = plsc.VectorSubcoreMesh(core_axis_name="c", subcore_axis_name="s")

  @functools.partial(
      pl.kernel,
      mesh=mesh,
      out_type=jax.ShapeDtypeStruct((NC, NP, DN), jnp.float32),
      scratch_types=[
          pltpu.VMEM((CH,), jnp.int32),
          pltpu.VMEM((CH,), jnp.int32),
          pltpu.VMEM((CH, DN), jnp.float32),
          pltpu.VMEM_SHARED((NP, DN), jnp.float32),
          pltpu.SemaphoreType.DMA,
      ],
  )
  def k(u_hbm, v_hbm, nf_hbm, sn_out, idx_u, idx_v, rows, acc_n, sem):
    cid = lax.axis_index("c")
    sid = lax.axis_index("s")
    w = sid * NC + cid

    _zero_2d(rows, CH, DN)
    base_r = sid * ROWS_PER_TILE
    for j in range(5):
      off = base_r + j * 128
      pltpu.sync_copy(rows, acc_n.at[pl.ds(off, 128)])
    plsc.subcore_barrier()

    def body(i, _):
      ch = w + i * NW

      @pl.when(ch < NCHUNK)
      def _():
        base = ch * CH
        pltpu.sync_copy(u_hbm.at[pl.ds(base, CH)], idx_u)
        pltpu.sync_copy(v_hbm.at[pl.ds(base, CH)], idx_v)
        pltpu.async_copy(nf_hbm.at[idx_u], rows, sem).wait()
        pltpu.sync_copy(rows, acc_n.at[idx_v], add=True)

      return 0

    lax.fori_loop(0, (NCHUNK + NW - 1) // NW, body, 0)
    plsc.subcore_barrier()

    for j in range(5):
      off = base_r + j * 128
      pltpu.sync_copy(acc_n.at[pl.ds(off, 128)],
                      sn_out.at[cid, pl.ds(off, 128)])

  return k(u, v, nfeats)


def _sc_pass_a2(v, efeats):
  mesh = plsc.VectorSubcoreMesh(core_axis_name="c", subcore_axis_name="s")

  @functools.partial(
      pl.kernel,
      mesh=mesh,
      out_type=jax.ShapeDtypeStruct((NC, NP, 128), jnp.float32),
      scratch_types=[
          pltpu.VMEM((CH,), jnp.int32),
          pltpu.VMEM((CH, DE), jnp.float32),
          pltpu.VMEM((CH, 128), jnp.float32),
          pltpu.VMEM_SHARED((NP, 128), jnp.float32),
      ],
  )
  def k(v_hbm, ef_hbm, sec_out, idx_v, ef_narrow, wide, acc):
    cid = lax.axis_index("c")
    sid = lax.axis_index("s")
    w = sid * NC + cid

    _zero_2d(wide, CH, 128)
    base_r = sid * ROWS_PER_TILE
    for j in range(5):
      off = base_r + j * 128
      pltpu.sync_copy(wide, acc.at[pl.ds(off, 128)])

    one = jnp.ones((16,), jnp.float32)

    def ones_body(r, _):
      wide[r, pl.ds(16, 16)] = one
      return 0

    lax.fori_loop(0, CH, ones_body, 0)
    plsc.subcore_barrier()

    def body(i, _):
      ch = w + i * NW

      @pl.when(ch < NCHUNK)
      def _():
        base = ch * CH
        pltpu.sync_copy(v_hbm.at[pl.ds(base, CH)], idx_v)
        pltpu.sync_copy(ef_hbm.at[pl.ds(base, CH)], ef_narrow)

        def cp(r, _):
          wide[r, pl.ds(0, 16)] = ef_narrow[r, pl.ds(0, 16)]
          return 0

        lax.fori_loop(0, CH, cp, 0)
        pltpu.sync_copy(wide, acc.at[idx_v], add=True)

      return 0

    lax.fori_loop(0, (NCHUNK + NW - 1) // NW, body, 0)
    plsc.subcore_barrier()

    for j in range(5):
      off = base_r + j * 128
      pltpu.sync_copy(acc.at[pl.ds(off, 128)],
                      sec_out.at[cid, pl.ds(off, 128)])

  return k(v, efeats)


def _sc_pass_b(u, v, h1, p1, q1):
  mesh = plsc.VectorSubcoreMesh(core_axis_name="c", subcore_axis_name="s")

  @functools.partial(
      pl.kernel,
      mesh=mesh,
      out_type=[
          jax.ShapeDtypeStruct((NP, H), jnp.float32),
          jax.ShapeDtypeStruct((NP, H), jnp.float32),
      ],
      scratch_types=[
          pltpu.VMEM((CH,), jnp.int32),
          pltpu.VMEM((CH,), jnp.int32),
          pltpu.VMEM((CH, H), jnp.float32),
          pltpu.VMEM((CH, H), jnp.float32),
          pltpu.VMEM_SHARED((NP, H), jnp.float32),
          pltpu.SemaphoreType.DMA,
      ],
  )
  def k(u_hbm, v_hbm, h1_hbm, p1_hbm, q1_hbm, sn_out, se_out,
        idx_u, idx_v, rows_a, rows_b, acc, sem):
    cid = lax.axis_index("c")
    sid = lax.axis_index("s")

    _zero_2d(rows_a, CH, H)
    base_r = sid * ROWS_PER_TILE
    for j in range(5):
      off = base_r + j * 128
      pltpu.sync_copy(rows_a, acc.at[pl.ds(off, 128)])
    plsc.subcore_barrier()

    nit = (NCHUNK + NS - 1) // NS

    @pl.when(cid == 0)
    def _core0():
      def body(i, _):
        ch = sid + i * NS

        @pl.when(ch < NCHUNK)
        def _():
          base = ch * CH
          pltpu.sync_copy(u_hbm.at[pl.ds(base, CH)], idx_u)
          pltpu.sync_copy(v_hbm.at[pl.ds(base, CH)], idx_v)
          pltpu.async_copy(h1_hbm.at[idx_u], rows_a, sem).wait()
          pltpu.sync_copy(rows_a, acc.at[idx_v], add=True)

        return 0

      lax.fori_loop(0, nit, body, 0)

    @pl.when(cid == 1)
    def _core1():
      def body(i, _):
        ch = sid + i * NS

        @pl.when(ch < NCHUNK)
        def _():
          base = ch * CH
          pltpu.sync_copy(u_hbm.at[pl.ds(base, CH)], idx_u)
          pltpu.sync_copy(v_hbm.at[pl.ds(base, CH)], idx_v)
          pltpu.async_copy(p1_hbm.at[idx_u], rows_a, sem).wait()
          pltpu.async_copy(q1_hbm.at[idx_v], rows_b, sem).wait()
          _relu_add_rows(rows_a, rows_b, CH, H)
          pltpu.sync_copy(rows_a, acc.at[idx_v], add=True)

        return 0

      lax.fori_loop(0, nit, body, 0)

    plsc.subcore_barrier()
    for j in range(5):
      off = base_r + j * 128

      @pl.when(cid == 0)
      def _():
        pltpu.sync_copy(acc.at[pl.ds(off, 128)], sn_out.at[pl.ds(off, 128)])

      @pl.when(cid == 1)
      def _():
        pltpu.sync_copy(acc.at[pl.ds(off, 128)], se_out.at[pl.ds(off, 128)])

  return k(u, v, h1, p1, q1)


def _sc_pass_c(u, v, p2, q2):
  mesh = plsc.VectorSubcoreMesh(core_axis_name="c", subcore_axis_name="s")

  @functools.partial(
      pl.kernel,
      mesh=mesh,
      out_type=jax.ShapeDtypeStruct((E, H), jnp.float32),
      scratch_types=[
          pltpu.VMEM((CH,), jnp.int32),
          pltpu.VMEM((CH,), jnp.int32),
          pltpu.VMEM((CH, H), jnp.float32),
          pltpu.VMEM((CH, H), jnp.float32),
          pltpu.SemaphoreType.DMA,
      ],
  )
  def k(u_hbm, v_hbm, p2_hbm, q2_hbm, e2_out,
        idx_u, idx_v, rows_a, rows_b, sem):
    cid = lax.axis_index("c")
    sid = lax.axis_index("s")
    w = sid * NC + cid

    def body(i, _):
      ch = w + i * NW

      @pl.when(ch < NCHUNK)
      def _():
        base = ch * CH
        pltpu.sync_copy(u_hbm.at[pl.ds(base, CH)], idx_u)
        pltpu.sync_copy(v_hbm.at[pl.ds(base, CH)], idx_v)
        pltpu.async_copy(p2_hbm.at[idx_u], rows_a, sem).wait()
        pltpu.async_copy(q2_hbm.at[idx_v], rows_b, sem).wait()
        _relu_add_rows(rows_a, rows_b, CH, H)
        pltpu.sync_copy(rows_a, e2_out.at[pl.ds(base, CH)])

      return 0

    lax.fori_loop(0, (NCHUNK + NW - 1) // NW, body, 0)

  return k(u, v, p2, q2)


_RB = 1024


def _tc_kernel_1(nfeats, sn_p, sec_p,
                 wan_t, wah_t, wae_t, ba, weu_t, wev_t, be):
  def body(nf, snp, secp, wan, wah, wae, b1, weu, wev, b2,
           h1o, p1o, q1o, cnto):
    sn = snp[0] + snp[1]
    sec = secp[0] + secp[1]
    se = sec[:, :DE]
    cnt = sec[:, DE:DE + 1]
    hn = jnp.where(cnt > 0, sn / jnp.maximum(cnt, 1.0), 0.0)
    he = jnp.where(cnt > 0, se / jnp.maximum(cnt, 1.0), 0.0)
    h = jnp.dot(nf[...], wan[...], preferred_element_type=jnp.float32)
    h += jnp.dot(hn, wah[...], preferred_element_type=jnp.float32)
    h += jnp.dot(he, wae[...], preferred_element_type=jnp.float32)
    h = jnp.maximum(h + b1[...], 0.0)
    h1o[...] = h
    p1o[...] = jnp.dot(h, weu[...], preferred_element_type=jnp.float32)
    q1o[...] = jnp.dot(h, wev[...], preferred_element_type=jnp.float32) + b2[...]
    cnto[...] = cnt + jnp.zeros((_RB, 16), jnp.float32)

  grid = (NP // _RB,)
  return pl.pallas_call(
      body,
      grid=grid,
      in_specs=[
          pl.BlockSpec((_RB, DN), lambda i: (i, 0)),
          pl.BlockSpec((NC, _RB, DN), lambda i: (0, i, 0)),
          pl.BlockSpec((NC, _RB, 128), lambda i: (0, i, 0)),
          pl.BlockSpec((DN, H), lambda i: (0, 0)),
          pl.BlockSpec((DN, H), lambda i: (0, 0)),
          pl.BlockSpec((DE, H), lambda i: (0, 0)),
          pl.BlockSpec((1, H), lambda i: (0, 0)),
          pl.BlockSpec((H, H), lambda i: (0, 0)),
          pl.BlockSpec((H, H), lambda i: (0, 0)),
          pl.BlockSpec((1, H), lambda i: (0, 0)),
      ],
      out_specs=[
          pl.BlockSpec((_RB, H), lambda i: (i, 0)),
          pl.BlockSpec((_RB, H), lambda i: (i, 0)),
          pl.BlockSpec((_RB, H), lambda i: (i, 0)),
          pl.BlockSpec((_RB, 16), lambda i: (i, 0)),
      ],
      out_shape=[
          jax.ShapeDtypeStruct((NP, H), jnp.float32),
          jax.ShapeDtypeStruct((NP, H), jnp.float32),
          jax.ShapeDtypeStruct((NP, H), jnp.float32),
          jax.ShapeDtypeStruct((NP, 16), jnp.float32),
      ],
  )(nfeats, sn_p, sec_p, wan_t, wah_t, wae_t, ba, weu_t, wev_t, be)


def _tc_kernel_2(h1, s2n, s2e, cnt,
                 wan_t, wah_t, wae_t, ba, weu_t, wev_t, be):
  def body(h1i, sn, se, cntp, wan, wah, wae, b1, weu, wev, b2,
           h2o, p2o, q2o):
    cnt1 = cntp[:, :1]
    hn = jnp.where(cnt1 > 0, sn[...] / jnp.maximum(cnt1, 1.0), 0.0)
    he = jnp.where(cnt1 > 0, se[...] / jnp.maximum(cnt1, 1.0), 0.0)
    h = jnp.dot(h1i[...], wan[...], preferred_element_type=jnp.float32)
    h += jnp.dot(hn, wah[...], preferred_element_type=jnp.float32)
    h += jnp.dot(he, wae[...], preferred_element_type=jnp.float32)
    h = jnp.maximum(h + b1[...], 0.0)
    h2o[...] = h
    p2o[...] = jnp.dot(h, weu[...], preferred_element_type=jnp.float32)
    q2o[...] = jnp.dot(h, wev[...], preferred_element_type=jnp.float32) + b2[...]

  grid = (NP // _RB,)
  return pl.pallas_call(
      body,
      grid=grid,
      in_specs=[
          pl.BlockSpec((_RB, H), lambda i: (i, 0)),
          pl.BlockSpec((_RB, H), lambda i: (i, 0)),
          pl.BlockSpec((_RB, H), lambda i: (i, 0)),
          pl.BlockSpec((_RB, 16), lambda i: (i, 0)),
          pl.BlockSpec((H, H), lambda i: (0, 0)),
          pl.BlockSpec((H, H), lambda i: (0, 0)),
          pl.BlockSpec((H, H), lambda i: (0, 0)),
          pl.BlockSpec((1, H), lambda i: (0, 0)),
          pl.BlockSpec((H, H), lambda i: (0, 0)),
          pl.BlockSpec((H, H), lambda i: (0, 0)),
          pl.BlockSpec((1, H), lambda i: (0, 0)),
      ],
      out_specs=[
          pl.BlockSpec((_RB, H), lambda i: (i, 0)),
          pl.BlockSpec((_RB, H), lambda i: (i, 0)),
          pl.BlockSpec((_RB, H), lambda i: (i, 0)),
      ],
      out_shape=[
          jax.ShapeDtypeStruct((NP, H), jnp.float32),
          jax.ShapeDtypeStruct((NP, H), jnp.float32),
          jax.ShapeDtypeStruct((NP, H), jnp.float32),
      ],
  )(h1, s2n, s2e, cnt, wan_t, wah_t, wae_t, ba, weu_t, wev_t, be)


def kernel(nfeats, edge_index, efeats, Wa1, ba1, We1, be1, Wa2, ba2, We2, be2):
  u = edge_index[0]
  v = edge_index[1]
  nfeats_p = jnp.pad(nfeats, ((0, NP - N), (0, 0)))

  sn_p = _sc_pass_a1(u, v, nfeats_p)
  sec_p = _sc_pass_a2(v, efeats)
  h1, p1, q1, cnt = _tc_kernel_1(
      nfeats_p, sn_p, sec_p,
      Wa1[:, :DN].T, Wa1[:, DN:2 * DN].T, Wa1[:, 2 * DN:].T,
      ba1.reshape(1, H),
      We1[:, :H].T, We1[:, H:].T, be1.reshape(1, H))

  s2n, s2e = _sc_pass_b(u, v, h1, p1, q1)
  h2, p2, q2 = _tc_kernel_2(
      h1, s2n, s2e, cnt,
      Wa2[:, :H].T, Wa2[:, H:2 * H].T, Wa2[:, 2 * H:].T,
      ba2.reshape(1, H),
      We2[:, :H].T, We2[:, H:].T, be2.reshape(1, H))

  e2 = _sc_pass_c(u, v, p2, q2)
  return (h2[:N], e2)


# speedup vs baseline: 5.2378x; 1.3721x over previous
import functools

import jax
import jax.numpy as jnp
from jax import lax
from jax.experimental import pallas as pl
from jax.experimental.pallas import tpu as pltpu
from jax.experimental.pallas import tpu_sc as plsc

N = 10000
NP = 10240
E = 320000
DN = 128
DE = 16
H = 128

CH = 128
CHB = 64
NCHUNK = E // CH
NCHUNKB = E // CHB
NC = 2
NS = 16
NW = NC * NS
ROWS_PER_TILE = NP // NS


def _zero_2d(ref, nrows, ncols):
  z = jnp.zeros((16,), jnp.float32)

  def body(r, _):
    for k in range(ncols // 16):
      ref[r, pl.ds(k * 16, 16)] = z
    return 0

  lax.fori_loop(0, nrows, body, 0)


def _zero_acc_slice(zrows, acc, sid):
  base_r = sid * ROWS_PER_TILE
  for j in range(5):
    pltpu.sync_copy(zrows, acc.at[pl.ds(base_r + j * 128, 128)])


def _writeout_acc(acc, out_ref, sid):
  base_r = sid * ROWS_PER_TILE
  for j in range(5):
    off = base_r + j * 128
    pltpu.sync_copy(acc.at[pl.ds(off, 128)], out_ref.at[pl.ds(off, 128)])


def _sc_pass_a(u, v, nfeats, efeats):
  mesh = plsc.VectorSubcoreMesh(core_axis_name="c", subcore_axis_name="s")
  nit = (NCHUNKB + NS - 1) // NS

  @functools.partial(
      pl.kernel,
      mesh=mesh,
      out_type=[
          jax.ShapeDtypeStruct((NP, DN), jnp.float32),
          jax.ShapeDtypeStruct((NP, 128), jnp.float32),
      ],
      scratch_types=[
          pltpu.VMEM((2, CHB), jnp.int32),
          pltpu.VMEM((2, CHB), jnp.int32),
          pltpu.VMEM((2, CHB, DE), jnp.float32),
          pltpu.VMEM((2, CHB, 128), jnp.float32),
          pltpu.VMEM_SHARED((NP, 128), jnp.float32),
          pltpu.SemaphoreType.DMA((2,)),
      ],
  )
  def k(u_hbm, v_hbm, nf_hbm, ef_hbm, sn_out, sec_out,
        idx_u, idx_v, ef_n, stage, acc, semg):
    cid = lax.axis_index("c")
    sid = lax.axis_index("s")

    _zero_2d(stage.at[0], CHB, 128)
    for j in range(10):
      pltpu.sync_copy(stage.at[0],
                      acc.at[pl.ds(sid * ROWS_PER_TILE + j * CHB, CHB)])

    @pl.when(cid == 1)
    def _():
      _zero_2d(stage.at[1], CHB, 128)
      one = jnp.ones((16,), jnp.float32)

      def ones_body(r, _):
        stage[0, r, pl.ds(DE, 16)] = one
        stage[1, r, pl.ds(DE, 16)] = one
        return 0

      lax.fori_loop(0, CHB, ones_body, 0)

    plsc.subcore_barrier()

    def chunk(g):
      return sid + g * NS

    def fetch(g, b):
      @pl.when(chunk(g) < NCHUNKB)
      def _():
        base = chunk(g) * CHB
        pltpu.sync_copy(v_hbm.at[pl.ds(base, CHB)], idx_v.at[b])

        @pl.when(cid == 0)
        def _():
          pltpu.sync_copy(u_hbm.at[pl.ds(base, CHB)], idx_u.at[b])
          pltpu.async_copy(nf_hbm.at[idx_u.at[b]], stage.at[b], semg.at[b])

        @pl.when(cid == 1)
        def _():
          pltpu.async_copy(ef_hbm.at[pl.ds(base, CHB)], ef_n.at[b],
                           semg.at[b])

    def process(g, b):
      @pl.when(chunk(g) < NCHUNKB)
      def _():
        @pl.when(cid == 0)
        def _():
          pltpu.make_async_copy(nf_hbm.at[idx_u.at[b]], stage.at[b],
                                semg.at[b]).wait()
          pltpu.sync_copy(stage.at[b], acc.at[idx_v.at[b]], add=True)

        @pl.when(cid == 1)
        def _():
          base = chunk(g) * CHB
          pltpu.make_async_copy(ef_hbm.at[pl.ds(base, CHB)], ef_n.at[b],
                                semg.at[b]).wait()

          def cp(r, _):
            stage[b, r, pl.ds(0, DE)] = ef_n[b, r, pl.ds(0, DE)]
            return 0

          lax.fori_loop(0, CHB, cp, 0)
          pltpu.sync_copy(stage.at[b], acc.at[idx_v.at[b]], add=True)

    fetch(0, 0)

    def body(gg, _):
      for b in range(2):
        g = 2 * gg + b
        fetch(g + 1, 1 - b)
        process(g, b)
      return 0

    lax.fori_loop(0, (nit + 1) // 2, body, 0)
    plsc.subcore_barrier()

    @pl.when(cid == 0)
    def _():
      _writeout_acc(acc, sn_out, sid)

    @pl.when(cid == 1)
    def _():
      _writeout_acc(acc, sec_out, sid)

  return k(u, v, nfeats, efeats)


def _sc_pass_b(u, v, h1, p1, q1):
  mesh = plsc.VectorSubcoreMesh(core_axis_name="c", subcore_axis_name="s")
  nit = (NCHUNKB + NS - 1) // NS

  @functools.partial(
      pl.kernel,
      mesh=mesh,
      out_type=[
          jax.ShapeDtypeStruct((NP, H), jnp.float32),
          jax.ShapeDtypeStruct((NP, H), jnp.float32),
      ],
      scratch_types=[
          pltpu.VMEM((2, CHB), jnp.int32),
          pltpu.VMEM((2, CHB), jnp.int32),
          pltpu.VMEM((2, CHB, H), jnp.float32),
          pltpu.VMEM((2, CHB, H), jnp.float32),
          pltpu.VMEM_SHARED((NP, H), jnp.float32),
          pltpu.SemaphoreType.DMA((2,)),
          pltpu.SemaphoreType.DMA((2,)),
      ],
  )
  def k(u_hbm, v_hbm, h1_hbm, p1_hbm, q1_hbm, sn_out, se_out,
        idx_u, idx_v, rows_a, rows_b, acc, sema, semb):
    cid = lax.axis_index("c")
    sid = lax.axis_index("s")

    _zero_2d(rows_a.at[0], CHB, H)
    _zero_2d(rows_a.at[1], CHB, H)
    for j in range(10):
      pltpu.sync_copy(rows_a.at[0],
                      acc.at[pl.ds(sid * ROWS_PER_TILE + j * CHB, CHB)])
    plsc.subcore_barrier()

    def chunk(g):
      return sid + g * NS

    def fetch(g, b):
      @pl.when(chunk(g) < NCHUNKB)
      def _():
        base = chunk(g) * CHB
        pltpu.sync_copy(v_hbm.at[pl.ds(base, CHB)], idx_v.at[b])
        pltpu.sync_copy(u_hbm.at[pl.ds(base, CHB)], idx_u.at[b])

        @pl.when(cid == 0)
        def _():
          pltpu.async_copy(h1_hbm.at[idx_u.at[b]], rows_a.at[b], sema.at[b])

        @pl.when(cid == 1)
        def _():
          pltpu.async_copy(p1_hbm.at[idx_u.at[b]], rows_a.at[b], sema.at[b])
          pltpu.async_copy(q1_hbm.at[idx_v.at[b]], rows_b.at[b], semb.at[b])

    def process(g, b):
      @pl.when(chunk(g) < NCHUNKB)
      def _():
        @pl.when(cid == 0)
        def _():
          pltpu.make_async_copy(h1_hbm.at[idx_u.at[b]], rows_a.at[b],
                                sema.at[b]).wait()
          pltpu.sync_copy(rows_a.at[b], acc.at[idx_v.at[b]], add=True)

        @pl.when(cid == 1)
        def _():
          pltpu.make_async_copy(p1_hbm.at[idx_u.at[b]], rows_a.at[b],
                                sema.at[b]).wait()
          pltpu.make_async_copy(q1_hbm.at[idx_v.at[b]], rows_b.at[b],
                                semb.at[b]).wait()

          def relu_add(r, _):
            for kk in range(H // 16):
              s = pl.ds(kk * 16, 16)
              rows_a[b, r, s] = jnp.maximum(rows_a[b, r, s] + rows_b[b, r, s],
                                            0.0)
            return 0

          lax.fori_loop(0, CHB, relu_add, 0)
          pltpu.sync_copy(rows_a.at[b], acc.at[idx_v.at[b]], add=True)

    fetch(0, 0)

    def body(gg, _):
      for b in range(2):
        g = 2 * gg + b
        fetch(g + 1, 1 - b)
        process(g, b)
      return 0

    lax.fori_loop(0, (nit + 1) // 2, body, 0)
    plsc.subcore_barrier()

    @pl.when(cid == 0)
    def _():
      _writeout_acc(acc, sn_out, sid)

    @pl.when(cid == 1)
    def _():
      _writeout_acc(acc, se_out, sid)

  return k(u, v, h1, p1, q1)


def _sc_pass_c(u, v, p2, q2):
  mesh = plsc.VectorSubcoreMesh(core_axis_name="c", subcore_axis_name="s")
  nit = (NCHUNK + NW - 1) // NW

  @functools.partial(
      pl.kernel,
      mesh=mesh,
      out_type=jax.ShapeDtypeStruct((E, H), jnp.float32),
      scratch_types=[
          pltpu.VMEM((2, CH), jnp.int32),
          pltpu.VMEM((2, CH), jnp.int32),
          pltpu.VMEM((2, CH, H), jnp.float32),
          pltpu.VMEM((2, CH, H), jnp.float32),
          pltpu.SemaphoreType.DMA((2,)),
          pltpu.SemaphoreType.DMA((2,)),
      ],
  )
  def k(u_hbm, v_hbm, p2_hbm, q2_hbm, e2_out,
        idx_u, idx_v, rows_a, rows_b, sema, semb):
    cid = lax.axis_index("c")
    sid = lax.axis_index("s")
    w = sid * NC + cid

    def chunk(g):
      return w + g * NW

    def fetch(g, b):
      @pl.when(chunk(g) < NCHUNK)
      def _():
        base = chunk(g) * CH
        pltpu.sync_copy(u_hbm.at[pl.ds(base, CH)], idx_u.at[b])
        pltpu.sync_copy(v_hbm.at[pl.ds(base, CH)], idx_v.at[b])
        pltpu.async_copy(p2_hbm.at[idx_u.at[b]], rows_a.at[b], sema.at[b])
        pltpu.async_copy(q2_hbm.at[idx_v.at[b]], rows_b.at[b], semb.at[b])

    def process(g, b):
      @pl.when(chunk(g) < NCHUNK)
      def _():
        pltpu.make_async_copy(p2_hbm.at[idx_u.at[b]], rows_a.at[b],
                              sema.at[b]).wait()
        pltpu.make_async_copy(q2_hbm.at[idx_v.at[b]], rows_b.at[b],
                              semb.at[b]).wait()

        def relu_add(r, _):
          for kk in range(H // 16):
            s = pl.ds(kk * 16, 16)
            rows_a[b, r, s] = jnp.maximum(rows_a[b, r, s] + rows_b[b, r, s],
                                          0.0)
          return 0

        lax.fori_loop(0, CH, relu_add, 0)
        base = chunk(g) * CH
        pltpu.sync_copy(rows_a.at[b], e2_out.at[pl.ds(base, CH)])

    fetch(0, 0)

    def body(gg, _):
      for b in range(2):
        g = 2 * gg + b
        fetch(g + 1, 1 - b)
        process(g, b)
      return 0

    lax.fori_loop(0, (nit + 1) // 2, body, 0)

  return k(u, v, p2, q2)


_RB = 1024


def _tc_kernel_1(nfeats, sn, sec,
                 wan_t, wah_t, wae_t, ba, weu_t, wev_t, be):
  def body(nf, snr, secr, wan, wah, wae, b1, weu, wev, b2,
           h1o, p1o, q1o, cnto):
    sn_ = snr[...]
    se = secr[:, :DE]
    cnt = secr[:, DE:DE + 1]
    hn = jnp.where(cnt > 0, sn_ / jnp.maximum(cnt, 1.0), 0.0)
    he = jnp.where(cnt > 0, se / jnp.maximum(cnt, 1.0), 0.0)
    h = jnp.dot(nf[...], wan[...], preferred_element_type=jnp.float32)
    h += jnp.dot(hn, wah[...], preferred_element_type=jnp.float32)
    h += jnp.dot(he, wae[...], preferred_element_type=jnp.float32)
    h = jnp.maximum(h + b1[...], 0.0)
    h1o[...] = h
    p1o[...] = jnp.dot(h, weu[...], preferred_element_type=jnp.float32)
    q1o[...] = jnp.dot(h, wev[...], preferred_element_type=jnp.float32) + b2[...]
    cnto[...] = cnt + jnp.zeros((_RB, 16), jnp.float32)

  grid = (NP // _RB,)
  return pl.pallas_call(
      body,
      grid=grid,
      in_specs=[
          pl.BlockSpec((_RB, DN), lambda i: (i, 0)),
          pl.BlockSpec((_RB, DN), lambda i: (i, 0)),
          pl.BlockSpec((_RB, 128), lambda i: (i, 0)),
          pl.BlockSpec((DN, H), lambda i: (0, 0)),
          pl.BlockSpec((DN, H), lambda i: (0, 0)),
          pl.BlockSpec((DE, H), lambda i: (0, 0)),
          pl.BlockSpec((1, H), lambda i: (0, 0)),
          pl.BlockSpec((H, H), lambda i: (0, 0)),
          pl.BlockSpec((H, H), lambda i: (0, 0)),
          pl.BlockSpec((1, H), lambda i: (0, 0)),
      ],
      out_specs=[
          pl.BlockSpec((_RB, H), lambda i: (i, 0)),
          pl.BlockSpec((_RB, H), lambda i: (i, 0)),
          pl.BlockSpec((_RB, H), lambda i: (i, 0)),
          pl.BlockSpec((_RB, 16), lambda i: (i, 0)),
      ],
      out_shape=[
          jax.ShapeDtypeStruct((NP, H), jnp.float32),
          jax.ShapeDtypeStruct((NP, H), jnp.float32),
          jax.ShapeDtypeStruct((NP, H), jnp.float32),
          jax.ShapeDtypeStruct((NP, 16), jnp.float32),
      ],
  )(nfeats, sn, sec, wan_t, wah_t, wae_t, ba, weu_t, wev_t, be)


def _tc_kernel_2(h1, s2n, s2e, cnt,
                 wan_t, wah_t, wae_t, ba, weu_t, wev_t, be):
  def body(h1i, sn, se, cntp, wan, wah, wae, b1, weu, wev, b2,
           h2o, p2o, q2o):
    cnt1 = cntp[:, :1]
    hn = jnp.where(cnt1 > 0, sn[...] / jnp.maximum(cnt1, 1.0), 0.0)
    he = jnp.where(cnt1 > 0, se[...] / jnp.maximum(cnt1, 1.0), 0.0)
    h = jnp.dot(h1i[...], wan[...], preferred_element_type=jnp.float32)
    h += jnp.dot(hn, wah[...], preferred_element_type=jnp.float32)
    h += jnp.dot(he, wae[...], preferred_element_type=jnp.float32)
    h = jnp.maximum(h + b1[...], 0.0)
    h2o[...] = h
    p2o[...] = jnp.dot(h, weu[...], preferred_element_type=jnp.float32)
    q2o[...] = jnp.dot(h, wev[...], preferred_element_type=jnp.float32) + b2[...]

  grid = (NP // _RB,)
  return pl.pallas_call(
      body,
      grid=grid,
      in_specs=[
          pl.BlockSpec((_RB, H), lambda i: (i, 0)),
          pl.BlockSpec((_RB, H), lambda i: (i, 0)),
          pl.BlockSpec((_RB, H), lambda i: (i, 0)),
          pl.BlockSpec((_RB, 16), lambda i: (i, 0)),
          pl.BlockSpec((H, H), lambda i: (0, 0)),
          pl.BlockSpec((H, H), lambda i: (0, 0)),
          pl.BlockSpec((H, H), lambda i: (0, 0)),
          pl.BlockSpec((1, H), lambda i: (0, 0)),
          pl.BlockSpec((H, H), lambda i: (0, 0)),
          pl.BlockSpec((H, H), lambda i: (0, 0)),
          pl.BlockSpec((1, H), lambda i: (0, 0)),
      ],
      out_specs=[
          pl.BlockSpec((_RB, H), lambda i: (i, 0)),
          pl.BlockSpec((_RB, H), lambda i: (i, 0)),
          pl.BlockSpec((_RB, H), lambda i: (i, 0)),
      ],
      out_shape=[
          jax.ShapeDtypeStruct((NP, H), jnp.float32),
          jax.ShapeDtypeStruct((NP, H), jnp.float32),
          jax.ShapeDtypeStruct((NP, H), jnp.float32),
      ],
  )(h1, s2n, s2e, cnt, wan_t, wah_t, wae_t, ba, weu_t, wev_t, be)


def kernel(nfeats, edge_index, efeats, Wa1, ba1, We1, be1, Wa2, ba2, We2, be2):
  u = edge_index[0]
  v = edge_index[1]
  nfeats_p = jnp.pad(nfeats, ((0, NP - N), (0, 0)))

  sn, sec = _sc_pass_a(u, v, nfeats_p, efeats)
  h1, p1, q1, cnt = _tc_kernel_1(
      nfeats_p, sn, sec,
      Wa1[:, :DN].T, Wa1[:, DN:2 * DN].T, Wa1[:, 2 * DN:].T,
      ba1.reshape(1, H),
      We1[:, :H].T, We1[:, H:].T, be1.reshape(1, H))

  s2n, s2e = _sc_pass_b(u, v, h1, p1, q1)
  h2, p2, q2 = _tc_kernel_2(
      h1, s2n, s2e, cnt,
      Wa2[:, :H].T, Wa2[:, H:2 * H].T, Wa2[:, 2 * H:].T,
      ba2.reshape(1, H),
      We2[:, :H].T, We2[:, H:].T, be2.reshape(1, H))

  e2 = _sc_pass_c(u, v, p2, q2)
  return (h2[:N], e2)


# speedup vs baseline: 5.5464x; 1.0589x over previous
import functools

import jax
import jax.numpy as jnp
from jax import lax
from jax.experimental import pallas as pl
from jax.experimental.pallas import tpu as pltpu
from jax.experimental.pallas import tpu_sc as plsc

N = 10000
NP = 10240
E = 320000
DN = 128
DE = 16
H = 128

CH = 128
CHB = 80
NCHUNK = E // CH
NCHUNKB = E // CHB
NC = 2
NS = 16
NW = NC * NS
ROWS_PER_TILE = NP // NS


def _zero_2d(ref, nrows, ncols):
  z = jnp.zeros((16,), jnp.float32)

  def body(r, _):
    for k in range(ncols // 16):
      ref[r, pl.ds(k * 16, 16)] = z
    return 0

  lax.fori_loop(0, nrows, body, 0)


def _writeout_acc(acc, out_ref, sid):
  base_r = sid * ROWS_PER_TILE
  for j in range(5):
    off = base_r + j * 128
    pltpu.sync_copy(acc.at[pl.ds(off, 128)], out_ref.at[pl.ds(off, 128)])


_EB = 8000


def _tc_kernel_0(efeats):
  def body(ef, out):
    out[...] = jnp.concatenate(
        [ef[...],
         jnp.ones((_EB, 16), jnp.float32),
         jnp.zeros((_EB, 96), jnp.float32)], axis=1)

  return pl.pallas_call(
      body,
      grid=(E // _EB,),
      in_specs=[pl.BlockSpec((_EB, DE), lambda i: (i, 0))],
      out_specs=pl.BlockSpec((_EB, 128), lambda i: (i, 0)),
      out_shape=jax.ShapeDtypeStruct((E, 128), jnp.float32),
  )(efeats)


def _sc_pass_a(u, v, nfeats, efp):
  mesh = plsc.VectorSubcoreMesh(core_axis_name="c", subcore_axis_name="s")
  nit = (NCHUNK + NS - 1) // NS

  @functools.partial(
      pl.kernel,
      mesh=mesh,
      out_type=[
          jax.ShapeDtypeStruct((NP, DN), jnp.float32),
          jax.ShapeDtypeStruct((NP, 128), jnp.float32),
      ],
      scratch_types=[
          pltpu.VMEM((2, CH), jnp.int32),
          pltpu.VMEM((2, CH), jnp.int32),
          pltpu.VMEM((2, CH, 128), jnp.float32),
          pltpu.VMEM_SHARED((NP, 128), jnp.float32),
          pltpu.SemaphoreType.DMA((2,)),
      ],
  )
  def k(u_hbm, v_hbm, nf_hbm, efp_hbm, sn_out, sec_out,
        idx_u, idx_v, stage, acc, semg):
    cid = lax.axis_index("c")
    sid = lax.axis_index("s")

    _zero_2d(stage.at[0], CH, 128)
    for j in range(5):
      pltpu.sync_copy(stage.at[0],
                      acc.at[pl.ds(sid * ROWS_PER_TILE + j * 128, 128)])
    plsc.subcore_barrier()

    def chunk(g):
      return sid + g * NS

    def fetch(g, b):
      @pl.when(chunk(g) < NCHUNK)
      def _():
        base = chunk(g) * CH
        pltpu.sync_copy(v_hbm.at[pl.ds(base, CH)], idx_v.at[b])

        @pl.when(cid == 0)
        def _():
          pltpu.sync_copy(u_hbm.at[pl.ds(base, CH)], idx_u.at[b])
          pltpu.async_copy(nf_hbm.at[idx_u.at[b]], stage.at[b], semg.at[b])

        @pl.when(cid == 1)
        def _():
          pltpu.async_copy(efp_hbm.at[pl.ds(base, CH)], stage.at[b],
                           semg.at[b])

    def process(g, b):
      @pl.when(chunk(g) < NCHUNK)
      def _():
        @pl.when(cid == 0)
        def _():
          pltpu.make_async_copy(nf_hbm.at[idx_u.at[b]], stage.at[b],
                                semg.at[b]).wait()

        @pl.when(cid == 1)
        def _():
          base = chunk(g) * CH
          pltpu.make_async_copy(efp_hbm.at[pl.ds(base, CH)], stage.at[b],
                                semg.at[b]).wait()

        pltpu.sync_copy(stage.at[b], acc.at[idx_v.at[b]], add=True)

    fetch(0, 0)

    def body(gg, _):
      for b in range(2):
        g = 2 * gg + b
        fetch(g + 1, 1 - b)
        process(g, b)
      return 0

    lax.fori_loop(0, (nit + 1) // 2, body, 0)
    plsc.subcore_barrier()

    @pl.when(cid == 0)
    def _():
      _writeout_acc(acc, sn_out, sid)

    @pl.when(cid == 1)
    def _():
      _writeout_acc(acc, sec_out, sid)

  return k(u, v, nfeats, efp)


def _sc_pass_b(u, v, h1, p1, q1):
  mesh = plsc.VectorSubcoreMesh(core_axis_name="c", subcore_axis_name="s")
  nit = (NCHUNKB + NS - 1) // NS

  @functools.partial(
      pl.kernel,
      mesh=mesh,
      out_type=[
          jax.ShapeDtypeStruct((NP, H), jnp.float32),
          jax.ShapeDtypeStruct((NP, H), jnp.float32),
      ],
      scratch_types=[
          pltpu.VMEM((2, CHB), jnp.int32),
          pltpu.VMEM((2, CHB), jnp.int32),
          pltpu.VMEM((2, CHB, H), jnp.float32),
          pltpu.VMEM((2, CHB, H), jnp.float32),
          pltpu.VMEM_SHARED((NP, H), jnp.float32),
          pltpu.SemaphoreType.DMA((2,)),
          pltpu.SemaphoreType.DMA((2,)),
      ],
  )
  def k(u_hbm, v_hbm, h1_hbm, p1_hbm, q1_hbm, sn_out, se_out,
        idx_u, idx_v, rows_a, rows_b, acc, sema, semb):
    cid = lax.axis_index("c")
    sid = lax.axis_index("s")

    _zero_2d(rows_a.at[0], CHB, H)
    for j in range(8):
      pltpu.sync_copy(rows_a.at[0],
                      acc.at[pl.ds(sid * ROWS_PER_TILE + j * CHB, CHB)])
    plsc.subcore_barrier()

    def chunk(g):
      return sid + g * NS

    def fetch(g, b):
      @pl.when(chunk(g) < NCHUNKB)
      def _():
        base = chunk(g) * CHB
        pltpu.sync_copy(v_hbm.at[pl.ds(base, CHB)], idx_v.at[b])
        pltpu.sync_copy(u_hbm.at[pl.ds(base, CHB)], idx_u.at[b])

        @pl.when(cid == 0)
        def _():
          pltpu.async_copy(h1_hbm.at[idx_u.at[b]], rows_a.at[b], sema.at[b])

        @pl.when(cid == 1)
        def _():
          pltpu.async_copy(p1_hbm.at[idx_u.at[b]], rows_a.at[b], sema.at[b])
          pltpu.async_copy(q1_hbm.at[idx_v.at[b]], rows_b.at[b], semb.at[b])

    def process(g, b):
      @pl.when(chunk(g) < NCHUNKB)
      def _():
        @pl.when(cid == 0)
        def _():
          pltpu.make_async_copy(h1_hbm.at[idx_u.at[b]], rows_a.at[b],
                                sema.at[b]).wait()

        @pl.when(cid == 1)
        def _():
          pltpu.make_async_copy(p1_hbm.at[idx_u.at[b]], rows_a.at[b],
                                sema.at[b]).wait()
          pltpu.make_async_copy(q1_hbm.at[idx_v.at[b]], rows_b.at[b],
                                semb.at[b]).wait()

          def relu_add(r, _):
            for kk in range(H // 16):
              s = pl.ds(kk * 16, 16)
              rows_a[b, r, s] = jnp.maximum(rows_a[b, r, s] + rows_b[b, r, s],
                                            0.0)
            return 0

          lax.fori_loop(0, CHB, relu_add, 0)

        pltpu.sync_copy(rows_a.at[b], acc.at[idx_v.at[b]], add=True)

    fetch(0, 0)

    def body(gg, _):
      for b in range(2):
        g = 2 * gg + b
        fetch(g + 1, 1 - b)
        process(g, b)
      return 0

    lax.fori_loop(0, (nit + 1) // 2, body, 0)
    plsc.subcore_barrier()

    @pl.when(cid == 0)
    def _():
      _writeout_acc(acc, sn_out, sid)

    @pl.when(cid == 1)
    def _():
      _writeout_acc(acc, se_out, sid)

  return k(u, v, h1, p1, q1)


def _sc_pass_c(u, v, p2, q2):
  mesh = plsc.VectorSubcoreMesh(core_axis_name="c", subcore_axis_name="s")
  nit = (NCHUNK + NW - 1) // NW

  @functools.partial(
      pl.kernel,
      mesh=mesh,
      out_type=jax.ShapeDtypeStruct((E, H), jnp.float32),
      scratch_types=[
          pltpu.VMEM((2, CH), jnp.int32),
          pltpu.VMEM((2, CH), jnp.int32),
          pltpu.VMEM((2, CH, H), jnp.float32),
          pltpu.VMEM((2, CH, H), jnp.float32),
          pltpu.SemaphoreType.DMA((2,)),
          pltpu.SemaphoreType.DMA((2,)),
      ],
  )
  def k(u_hbm, v_hbm, p2_hbm, q2_hbm, e2_out,
        idx_u, idx_v, rows_a, rows_b, sema, semb):
    cid = lax.axis_index("c")
    sid = lax.axis_index("s")
    w = sid * NC + cid

    def chunk(g):
      return w + g * NW

    def fetch(g, b):
      @pl.when(chunk(g) < NCHUNK)
      def _():
        base = chunk(g) * CH
        pltpu.sync_copy(u_hbm.at[pl.ds(base, CH)], idx_u.at[b])
        pltpu.sync_copy(v_hbm.at[pl.ds(base, CH)], idx_v.at[b])
        pltpu.async_copy(p2_hbm.at[idx_u.at[b]], rows_a.at[b], sema.at[b])
        pltpu.async_copy(q2_hbm.at[idx_v.at[b]], rows_b.at[b], semb.at[b])

    def process(g, b):
      @pl.when(chunk(g) < NCHUNK)
      def _():
        pltpu.make_async_copy(p2_hbm.at[idx_u.at[b]], rows_a.at[b],
                              sema.at[b]).wait()
        pltpu.make_async_copy(q2_hbm.at[idx_v.at[b]], rows_b.at[b],
                              semb.at[b]).wait()

        def relu_add(r, _):
          for kk in range(H // 16):
            s = pl.ds(kk * 16, 16)
            rows_a[b, r, s] = jnp.maximum(rows_a[b, r, s] + rows_b[b, r, s],
                                          0.0)
          return 0

        lax.fori_loop(0, CH, relu_add, 0)
        base = chunk(g) * CH
        pltpu.sync_copy(rows_a.at[b], e2_out.at[pl.ds(base, CH)])

    fetch(0, 0)

    def body(gg, _):
      for b in range(2):
        g = 2 * gg + b
        fetch(g + 1, 1 - b)
        process(g, b)
      return 0

    lax.fori_loop(0, (nit + 1) // 2, body, 0)

  return k(u, v, p2, q2)


_RB = 1024


def _tc_kernel_1(nfeats, sn, sec,
                 wan_t, wah_t, wae_t, ba, weu_t, wev_t, be):
  def body(nf, snr, secr, wan, wah, wae, b1, weu, wev, b2,
           h1o, p1o, q1o, cnto):
    sn_ = snr[...]
    se = secr[:, :DE]
    cnt = secr[:, DE:DE + 1]
    hn = jnp.where(cnt > 0, sn_ / jnp.maximum(cnt, 1.0), 0.0)
    he = jnp.where(cnt > 0, se / jnp.maximum(cnt, 1.0), 0.0)
    h = jnp.dot(nf[...], wan[...], preferred_element_type=jnp.float32)
    h += jnp.dot(hn, wah[...], preferred_element_type=jnp.float32)
    h += jnp.dot(he, wae[...], preferred_element_type=jnp.float32)
    h = jnp.maximum(h + b1[...], 0.0)
    h1o[...] = h
    p1o[...] = jnp.dot(h, weu[...], preferred_element_type=jnp.float32)
    q1o[...] = jnp.dot(h, wev[...], preferred_element_type=jnp.float32) + b2[...]
    cnto[...] = cnt + jnp.zeros((_RB, 16), jnp.float32)

  grid = (NP // _RB,)
  return pl.pallas_call(
      body,
      grid=grid,
      in_specs=[
          pl.BlockSpec((_RB, DN), lambda i: (i, 0)),
          pl.BlockSpec((_RB, DN), lambda i: (i, 0)),
          pl.BlockSpec((_RB, 128), lambda i: (i, 0)),
          pl.BlockSpec((DN, H), lambda i: (0, 0)),
          pl.BlockSpec((DN, H), lambda i: (0, 0)),
          pl.BlockSpec((DE, H), lambda i: (0, 0)),
          pl.BlockSpec((1, H), lambda i: (0, 0)),
          pl.BlockSpec((H, H), lambda i: (0, 0)),
          pl.BlockSpec((H, H), lambda i: (0, 0)),
          pl.BlockSpec((1, H), lambda i: (0, 0)),
      ],
      out_specs=[
          pl.BlockSpec((_RB, H), lambda i: (i, 0)),
          pl.BlockSpec((_RB, H), lambda i: (i, 0)),
          pl.BlockSpec((_RB, H), lambda i: (i, 0)),
          pl.BlockSpec((_RB, 16), lambda i: (i, 0)),
      ],
      out_shape=[
          jax.ShapeDtypeStruct((NP, H), jnp.float32),
          jax.ShapeDtypeStruct((NP, H), jnp.float32),
          jax.ShapeDtypeStruct((NP, H), jnp.float32),
          jax.ShapeDtypeStruct((NP, 16), jnp.float32),
      ],
  )(nfeats, sn, sec, wan_t, wah_t, wae_t, ba, weu_t, wev_t, be)


def _tc_kernel_2(h1, s2n, s2e, cnt,
                 wan_t, wah_t, wae_t, ba, weu_t, wev_t, be):
  def body(h1i, sn, se, cntp, wan, wah, wae, b1, weu, wev, b2,
           h2o, p2o, q2o):
    cnt1 = cntp[:, :1]
    hn = jnp.where(cnt1 > 0, sn[...] / jnp.maximum(cnt1, 1.0), 0.0)
    he = jnp.where(cnt1 > 0, se[...] / jnp.maximum(cnt1, 1.0), 0.0)
    h = jnp.dot(h1i[...], wan[...], preferred_element_type=jnp.float32)
    h += jnp.dot(hn, wah[...], preferred_element_type=jnp.float32)
    h += jnp.dot(he, wae[...], preferred_element_type=jnp.float32)
    h = jnp.maximum(h + b1[...], 0.0)
    h2o[...] = h
    p2o[...] = jnp.dot(h, weu[...], preferred_element_type=jnp.float32)
    q2o[...] = jnp.dot(h, wev[...], preferred_element_type=jnp.float32) + b2[...]

  grid = (NP // _RB,)
  return pl.pallas_call(
      body,
      grid=grid,
      in_specs=[
          pl.BlockSpec((_RB, H), lambda i: (i, 0)),
          pl.BlockSpec((_RB, H), lambda i: (i, 0)),
          pl.BlockSpec((_RB, H), lambda i: (i, 0)),
          pl.BlockSpec((_RB, 16), lambda i: (i, 0)),
          pl.BlockSpec((H, H), lambda i: (0, 0)),
          pl.BlockSpec((H, H), lambda i: (0, 0)),
          pl.BlockSpec((H, H), lambda i: (0, 0)),
          pl.BlockSpec((1, H), lambda i: (0, 0)),
          pl.BlockSpec((H, H), lambda i: (0, 0)),
          pl.BlockSpec((H, H), lambda i: (0, 0)),
          pl.BlockSpec((1, H), lambda i: (0, 0)),
      ],
      out_specs=[
          pl.BlockSpec((_RB, H), lambda i: (i, 0)),
          pl.BlockSpec((_RB, H), lambda i: (i, 0)),
          pl.BlockSpec((_RB, H), lambda i: (i, 0)),
      ],
      out_shape=[
          jax.ShapeDtypeStruct((NP, H), jnp.float32),
          jax.ShapeDtypeStruct((NP, H), jnp.float32),
          jax.ShapeDtypeStruct((NP, H), jnp.float32),
      ],
  )(h1, s2n, s2e, cnt, wan_t, wah_t, wae_t, ba, weu_t, wev_t, be)


def kernel(nfeats, edge_index, efeats, Wa1, ba1, We1, be1, Wa2, ba2, We2, be2):
  u = edge_index[0]
  v = edge_index[1]
  nfeats_p = jnp.pad(nfeats, ((0, NP - N), (0, 0)))

  efp = _tc_kernel_0(efeats)
  sn, sec = _sc_pass_a(u, v, nfeats_p, efp)
  h1, p1, q1, cnt = _tc_kernel_1(
      nfeats_p, sn, sec,
      Wa1[:, :DN].T, Wa1[:, DN:2 * DN].T, Wa1[:, 2 * DN:].T,
      ba1.reshape(1, H),
      We1[:, :H].T, We1[:, H:].T, be1.reshape(1, H))

  s2n, s2e = _sc_pass_b(u, v, h1, p1, q1)
  h2, p2, q2 = _tc_kernel_2(
      h1, s2n, s2e, cnt,
      Wa2[:, :H].T, Wa2[:, H:2 * H].T, Wa2[:, 2 * H:].T,
      ba2.reshape(1, H),
      We2[:, :H].T, We2[:, H:].T, be2.reshape(1, H))

  e2 = _sc_pass_c(u, v, p2, q2)
  return (h2[:N], e2)


# speedup vs baseline: 5.9878x; 1.0796x over previous
import functools

import jax
import jax.numpy as jnp
from jax import lax
from jax.experimental import pallas as pl
from jax.experimental.pallas import tpu as pltpu
from jax.experimental.pallas import tpu_sc as plsc

N = 10000
NP = 10240
E = 320000
DN = 128
DE = 16
H = 128

CH = 128
CHB = 80
NCHUNK = E // CH
NCHUNKB = E // CHB
NC = 2
NS = 16
NW = NC * NS
ROWS_PER_TILE = NP // NS


def _zero_2d(ref, nrows, ncols):
  z = jnp.zeros((16,), jnp.float32)

  def body(r, _):
    for k in range(ncols // 16):
      ref[r, pl.ds(k * 16, 16)] = z
    return 0

  lax.fori_loop(0, nrows, body, 0)


def _writeout_acc(acc, out_ref, sid):
  base_r = sid * ROWS_PER_TILE
  for j in range(5):
    off = base_r + j * 128
    pltpu.sync_copy(acc.at[pl.ds(off, 128)], out_ref.at[pl.ds(off, 128)])


_EB = 8000


def _tc_kernel_0(efeats):
  def body(ef, out):
    out[...] = jnp.concatenate(
        [ef[...],
         jnp.ones((_EB, 16), jnp.float32),
         jnp.zeros((_EB, 96), jnp.float32)], axis=1)

  return pl.pallas_call(
      body,
      grid=(E // _EB,),
      in_specs=[pl.BlockSpec((_EB, DE), lambda i: (i, 0))],
      out_specs=pl.BlockSpec((_EB, 128), lambda i: (i, 0)),
      out_shape=jax.ShapeDtypeStruct((E, 128), jnp.float32),
  )(efeats)


def _sc_pass_a(uv, nfeats, efp):
  mesh = plsc.VectorSubcoreMesh(core_axis_name="c", subcore_axis_name="s")
  nit = (NCHUNK + NS - 1) // NS

  @functools.partial(
      pl.kernel,
      mesh=mesh,
      out_type=[
          jax.ShapeDtypeStruct((NP, DN), jnp.float32),
          jax.ShapeDtypeStruct((NP, 128), jnp.float32),
      ],
      scratch_types=[
          pltpu.VMEM((2, 2, CH), jnp.int32),
          pltpu.VMEM((2, CH, 128), jnp.float32),
          pltpu.VMEM_SHARED((NP, 128), jnp.float32),
          pltpu.SemaphoreType.DMA((2,)),
      ],
  )
  def k(uv_hbm, nf_hbm, efp_hbm, sn_out, sec_out,
        idx, stage, acc, semg):
    cid = lax.axis_index("c")
    sid = lax.axis_index("s")

    _zero_2d(stage.at[0], CH, 128)
    for j in range(5):
      pltpu.sync_copy(stage.at[0],
                      acc.at[pl.ds(sid * ROWS_PER_TILE + j * 128, 128)])
    plsc.subcore_barrier()

    def chunk(g):
      return sid + g * NS

    def fetch(g, b):
      @pl.when(chunk(g) < NCHUNK)
      def _():
        base = chunk(g) * CH
        pltpu.sync_copy(uv_hbm.at[:, pl.ds(base, CH)], idx.at[b])

        @pl.when(cid == 0)
        def _():
          pltpu.async_copy(nf_hbm.at[idx.at[b, 0]], stage.at[b], semg.at[b])

        @pl.when(cid == 1)
        def _():
          pltpu.async_copy(efp_hbm.at[pl.ds(base, CH)], stage.at[b],
                           semg.at[b])

    def process(g, b):
      @pl.when(chunk(g) < NCHUNK)
      def _():
        @pl.when(cid == 0)
        def _():
          pltpu.make_async_copy(nf_hbm.at[idx.at[b, 0]], stage.at[b],
                                semg.at[b]).wait()

        @pl.when(cid == 1)
        def _():
          base = chunk(g) * CH
          pltpu.make_async_copy(efp_hbm.at[pl.ds(base, CH)], stage.at[b],
                                semg.at[b]).wait()

        pltpu.sync_copy(stage.at[b], acc.at[idx.at[b, 1]], add=True)

    fetch(0, 0)

    def body(gg, _):
      for b in range(2):
        g = 2 * gg + b
        fetch(g + 1, 1 - b)
        process(g, b)
      return 0

    lax.fori_loop(0, (nit + 1) // 2, body, 0)
    plsc.subcore_barrier()

    @pl.when(cid == 0)
    def _():
      _writeout_acc(acc, sn_out, sid)

    @pl.when(cid == 1)
    def _():
      _writeout_acc(acc, sec_out, sid)

  return k(uv, nfeats, efp)


def _sc_pass_b(u, v, h1, p1, q1):
  mesh = plsc.VectorSubcoreMesh(core_axis_name="c", subcore_axis_name="s")
  nit = (NCHUNKB + NS - 1) // NS

  @functools.partial(
      pl.kernel,
      mesh=mesh,
      out_type=[
          jax.ShapeDtypeStruct((NP, H), jnp.float32),
          jax.ShapeDtypeStruct((NP, H), jnp.float32),
      ],
      scratch_types=[
          pltpu.VMEM((2, CHB), jnp.int32),
          pltpu.VMEM((2, CHB), jnp.int32),
          pltpu.VMEM((2, CHB, H), jnp.float32),
          pltpu.VMEM((2, CHB, H), jnp.float32),
          pltpu.VMEM_SHARED((NP, H), jnp.float32),
          pltpu.SemaphoreType.DMA((2,)),
          pltpu.SemaphoreType.DMA((2,)),
      ],
  )
  def k(u_hbm, v_hbm, h1_hbm, p1_hbm, q1_hbm, sn_out, se_out,
        idx_u, idx_v, rows_a, rows_b, acc, sema, semb):
    cid = lax.axis_index("c")
    sid = lax.axis_index("s")

    _zero_2d(rows_a.at[0], CHB, H)
    for j in range(8):
      pltpu.sync_copy(rows_a.at[0],
                      acc.at[pl.ds(sid * ROWS_PER_TILE + j * CHB, CHB)])
    plsc.subcore_barrier()

    def chunk(g):
      return sid + g * NS

    def fetch(g, b):
      @pl.when(chunk(g) < NCHUNKB)
      def _():
        base = chunk(g) * CHB
        pltpu.sync_copy(v_hbm.at[pl.ds(base, CHB)], idx_v.at[b])
        pltpu.sync_copy(u_hbm.at[pl.ds(base, CHB)], idx_u.at[b])

        @pl.when(cid == 0)
        def _():
          pltpu.async_copy(h1_hbm.at[idx_u.at[b]], rows_a.at[b], sema.at[b])

        @pl.when(cid == 1)
        def _():
          pltpu.async_copy(p1_hbm.at[idx_u.at[b]], rows_a.at[b], sema.at[b])
          pltpu.async_copy(q1_hbm.at[idx_v.at[b]], rows_b.at[b], semb.at[b])

    def process(g, b):
      @pl.when(chunk(g) < NCHUNKB)
      def _():
        @pl.when(cid == 0)
        def _():
          pltpu.make_async_copy(h1_hbm.at[idx_u.at[b]], rows_a.at[b],
                                sema.at[b]).wait()

        @pl.when(cid == 1)
        def _():
          pltpu.make_async_copy(p1_hbm.at[idx_u.at[b]], rows_a.at[b],
                                sema.at[b]).wait()
          pltpu.make_async_copy(q1_hbm.at[idx_v.at[b]], rows_b.at[b],
                                semb.at[b]).wait()

          def relu_add(r, _):
            for kk in range(H // 16):
              s = pl.ds(kk * 16, 16)
              rows_a[b, r, s] = jnp.maximum(rows_a[b, r, s] + rows_b[b, r, s],
                                            0.0)
            return 0

          lax.fori_loop(0, CHB, relu_add, 0)

        pltpu.sync_copy(rows_a.at[b], acc.at[idx_v.at[b]], add=True)

    fetch(0, 0)

    def body(gg, _):
      for b in range(2):
        g = 2 * gg + b
        fetch(g + 1, 1 - b)
        process(g, b)
      return 0

    lax.fori_loop(0, (nit + 1) // 2, body, 0)
    plsc.subcore_barrier()

    @pl.when(cid == 0)
    def _():
      _writeout_acc(acc, sn_out, sid)

    @pl.when(cid == 1)
    def _():
      _writeout_acc(acc, se_out, sid)

  return k(u, v, h1, p1, q1)


def _sc_pass_c(uv, p2, q2):
  mesh = plsc.VectorSubcoreMesh(core_axis_name="c", subcore_axis_name="s")
  nit = (NCHUNK + NW - 1) // NW

  @functools.partial(
      pl.kernel,
      mesh=mesh,
      out_type=jax.ShapeDtypeStruct((E, H), jnp.float32),
      scratch_types=[
          pltpu.VMEM((2, 2, CH), jnp.int32),
          pltpu.VMEM((2, CH, H), jnp.float32),
          pltpu.VMEM((2, CH, H), jnp.float32),
          pltpu.SemaphoreType.DMA((2,)),
          pltpu.SemaphoreType.DMA((2,)),
      ],
  )
  def k(uv_hbm, p2_hbm, q2_hbm, e2_out,
        idx, rows_a, rows_b, sema, semb):
    cid = lax.axis_index("c")
    sid = lax.axis_index("s")
    w = sid * NC + cid

    def chunk(g):
      return w + g * NW

    def fetch(g, b):
      @pl.when(chunk(g) < NCHUNK)
      def _():
        base = chunk(g) * CH
        pltpu.sync_copy(uv_hbm.at[:, pl.ds(base, CH)], idx.at[b])
        pltpu.async_copy(p2_hbm.at[idx.at[b, 0]], rows_a.at[b], sema.at[b])
        pltpu.async_copy(q2_hbm.at[idx.at[b, 1]], rows_b.at[b], semb.at[b])

    def process(g, b):
      @pl.when(chunk(g) < NCHUNK)
      def _():
        pltpu.make_async_copy(p2_hbm.at[idx.at[b, 0]], rows_a.at[b],
                              sema.at[b]).wait()
        pltpu.make_async_copy(q2_hbm.at[idx.at[b, 1]], rows_b.at[b],
                              semb.at[b]).wait()

        def relu_add(r, _):
          for kk in range(H // 16):
            s = pl.ds(kk * 16, 16)
            rows_a[b, r, s] = jnp.maximum(rows_a[b, r, s] + rows_b[b, r, s],
                                          0.0)
          return 0

        lax.fori_loop(0, CH, relu_add, 0)
        base = chunk(g) * CH
        pltpu.sync_copy(rows_a.at[b], e2_out.at[pl.ds(base, CH)])

    fetch(0, 0)

    def body(gg, _):
      for b in range(2):
        g = 2 * gg + b
        fetch(g + 1, 1 - b)
        process(g, b)
      return 0

    lax.fori_loop(0, (nit + 1) // 2, body, 0)

  return k(uv, p2, q2)


_RB = 1024


def _tc_kernel_1(nfeats, sn, sec,
                 wan_t, wah_t, wae_t, ba, weu_t, wev_t, be):
  def body(nf, snr, secr, wan, wah, wae, b1, weu, wev, b2,
           h1o, p1o, q1o, cnto):
    sn_ = snr[...]
    se = secr[:, :DE]
    cnt = secr[:, DE:DE + 1]
    hn = jnp.where(cnt > 0, sn_ / jnp.maximum(cnt, 1.0), 0.0)
    he = jnp.where(cnt > 0, se / jnp.maximum(cnt, 1.0), 0.0)
    h = jnp.dot(nf[...], wan[...], preferred_element_type=jnp.float32)
    h += jnp.dot(hn, wah[...], preferred_element_type=jnp.float32)
    h += jnp.dot(he, wae[...], preferred_element_type=jnp.float32)
    h = jnp.maximum(h + b1[...], 0.0)
    h1o[...] = h
    p1o[...] = jnp.dot(h, weu[...], preferred_element_type=jnp.float32)
    q1o[...] = jnp.dot(h, wev[...], preferred_element_type=jnp.float32) + b2[...]
    cnto[...] = cnt + jnp.zeros((_RB, 16), jnp.float32)

  grid = (NP // _RB,)
  return pl.pallas_call(
      body,
      grid=grid,
      in_specs=[
          pl.BlockSpec((_RB, DN), lambda i: (i, 0)),
          pl.BlockSpec((_RB, DN), lambda i: (i, 0)),
          pl.BlockSpec((_RB, 128), lambda i: (i, 0)),
          pl.BlockSpec((DN, H), lambda i: (0, 0)),
          pl.BlockSpec((DN, H), lambda i: (0, 0)),
          pl.BlockSpec((DE, H), lambda i: (0, 0)),
          pl.BlockSpec((1, H), lambda i: (0, 0)),
          pl.BlockSpec((H, H), lambda i: (0, 0)),
          pl.BlockSpec((H, H), lambda i: (0, 0)),
          pl.BlockSpec((1, H), lambda i: (0, 0)),
      ],
      out_specs=[
          pl.BlockSpec((_RB, H), lambda i: (i, 0)),
          pl.BlockSpec((_RB, H), lambda i: (i, 0)),
          pl.BlockSpec((_RB, H), lambda i: (i, 0)),
          pl.BlockSpec((_RB, 16), lambda i: (i, 0)),
      ],
      out_shape=[
          jax.ShapeDtypeStruct((NP, H), jnp.float32),
          jax.ShapeDtypeStruct((NP, H), jnp.float32),
          jax.ShapeDtypeStruct((NP, H), jnp.float32),
          jax.ShapeDtypeStruct((NP, 16), jnp.float32),
      ],
  )(nfeats, sn, sec, wan_t, wah_t, wae_t, ba, weu_t, wev_t, be)


def _tc_kernel_2(h1, s2n, s2e, cnt,
                 wan_t, wah_t, wae_t, ba, weu_t, wev_t, be):
  def body(h1i, sn, se, cntp, wan, wah, wae, b1, weu, wev, b2,
           h2o, p2o, q2o):
    cnt1 = cntp[:, :1]
    hn = jnp.where(cnt1 > 0, sn[...] / jnp.maximum(cnt1, 1.0), 0.0)
    he = jnp.where(cnt1 > 0, se[...] / jnp.maximum(cnt1, 1.0), 0.0)
    h = jnp.dot(h1i[...], wan[...], preferred_element_type=jnp.float32)
    h += jnp.dot(hn, wah[...], preferred_element_type=jnp.float32)
    h += jnp.dot(he, wae[...], preferred_element_type=jnp.float32)
    h = jnp.maximum(h + b1[...], 0.0)
    h2o[...] = h
    p2o[...] = jnp.dot(h, weu[...], preferred_element_type=jnp.float32)
    q2o[...] = jnp.dot(h, wev[...], preferred_element_type=jnp.float32) + b2[...]

  grid = (NP // _RB,)
  return pl.pallas_call(
      body,
      grid=grid,
      in_specs=[
          pl.BlockSpec((_RB, H), lambda i: (i, 0)),
          pl.BlockSpec((_RB, H), lambda i: (i, 0)),
          pl.BlockSpec((_RB, H), lambda i: (i, 0)),
          pl.BlockSpec((_RB, 16), lambda i: (i, 0)),
          pl.BlockSpec((H, H), lambda i: (0, 0)),
          pl.BlockSpec((H, H), lambda i: (0, 0)),
          pl.BlockSpec((H, H), lambda i: (0, 0)),
          pl.BlockSpec((1, H), lambda i: (0, 0)),
          pl.BlockSpec((H, H), lambda i: (0, 0)),
          pl.BlockSpec((H, H), lambda i: (0, 0)),
          pl.BlockSpec((1, H), lambda i: (0, 0)),
      ],
      out_specs=[
          pl.BlockSpec((_RB, H), lambda i: (i, 0)),
          pl.BlockSpec((_RB, H), lambda i: (i, 0)),
          pl.BlockSpec((_RB, H), lambda i: (i, 0)),
      ],
      out_shape=[
          jax.ShapeDtypeStruct((NP, H), jnp.float32),
          jax.ShapeDtypeStruct((NP, H), jnp.float32),
          jax.ShapeDtypeStruct((NP, H), jnp.float32),
      ],
  )(h1, s2n, s2e, cnt, wan_t, wah_t, wae_t, ba, weu_t, wev_t, be)


def kernel(nfeats, edge_index, efeats, Wa1, ba1, We1, be1, Wa2, ba2, We2, be2):
  uv = edge_index
  nfeats_p = jnp.pad(nfeats, ((0, NP - N), (0, 0)))

  efp = _tc_kernel_0(efeats)
  sn, sec = _sc_pass_a(uv, nfeats_p, efp)
  h1, p1, q1, cnt = _tc_kernel_1(
      nfeats_p, sn, sec,
      Wa1[:, :DN].T, Wa1[:, DN:2 * DN].T, Wa1[:, 2 * DN:].T,
      ba1.reshape(1, H),
      We1[:, :H].T, We1[:, H:].T, be1.reshape(1, H))

  s2n, s2e = _sc_pass_b(uv[0], uv[1], h1, p1, q1)
  h2, p2, q2 = _tc_kernel_2(
      h1, s2n, s2e, cnt,
      Wa2[:, :H].T, Wa2[:, H:2 * H].T, Wa2[:, 2 * H:].T,
      ba2.reshape(1, H),
      We2[:, :H].T, We2[:, H:].T, be2.reshape(1, H))

  e2 = _sc_pass_c(uv, p2, q2)
  return (h2[:N], e2)


# speedup vs baseline: 6.4807x; 1.0823x over previous
import functools

import jax
import jax.numpy as jnp
from jax import lax
from jax.experimental import pallas as pl
from jax.experimental.pallas import tpu as pltpu
from jax.experimental.pallas import tpu_sc as plsc

N = 10000
NP = 10240
E = 320000
DN = 128
DE = 16
H = 128

CH = 128
CHB = 80
NCHUNK = E // CH
NCHUNKB = E // CHB
NC = 2
NS = 16
NW = NC * NS
ROWS_PER_TILE = NP // NS


def _zero_2d(ref, nrows, ncols):
  z = jnp.zeros((16,), jnp.float32)

  def body(r, _):
    for k in range(ncols // 16):
      ref[r, pl.ds(k * 16, 16)] = z
    return 0

  lax.fori_loop(0, nrows, body, 0)


def _writeout_acc(acc, out_ref, sid):
  base_r = sid * ROWS_PER_TILE
  for j in range(5):
    off = base_r + j * 128
    pltpu.sync_copy(acc.at[pl.ds(off, 128)], out_ref.at[pl.ds(off, 128)])


_EB = 8000


def _tc_kernel_0(efeats):
  def body(ef, out):
    out[...] = jnp.concatenate(
        [ef[...],
         jnp.ones((_EB, 16), jnp.float32),
         jnp.zeros((_EB, 96), jnp.float32)], axis=1)

  return pl.pallas_call(
      body,
      grid=(E // _EB,),
      in_specs=[pl.BlockSpec((_EB, DE), lambda i: (i, 0))],
      out_specs=pl.BlockSpec((_EB, 128), lambda i: (i, 0)),
      out_shape=jax.ShapeDtypeStruct((E, 128), jnp.float32),
  )(efeats)


def _sc_pass_a(uv, nfeats, efp):
  mesh = plsc.VectorSubcoreMesh(core_axis_name="c", subcore_axis_name="s")
  nit = (NCHUNK + NS - 1) // NS

  @functools.partial(
      pl.kernel,
      mesh=mesh,
      out_type=[
          jax.ShapeDtypeStruct((NP, DN), jnp.float32),
          jax.ShapeDtypeStruct((NP, 128), jnp.float32),
      ],
      scratch_types=[
          pltpu.VMEM((2, 2, CH), jnp.int32),
          pltpu.VMEM((2, CH, 128), jnp.float32),
          pltpu.VMEM_SHARED((NP, 128), jnp.float32),
          pltpu.SemaphoreType.DMA((2,)),
      ],
  )
  def k(uv_hbm, nf_hbm, efp_hbm, sn_out, sec_out,
        idx, stage, acc, semg):
    cid = lax.axis_index("c")
    sid = lax.axis_index("s")

    _zero_2d(stage.at[0], CH, 128)
    for j in range(5):
      pltpu.sync_copy(stage.at[0],
                      acc.at[pl.ds(sid * ROWS_PER_TILE + j * 128, 128)])
    plsc.subcore_barrier()

    def chunk(g):
      return sid + g * NS

    def fetch(g, b):
      @pl.when(chunk(g) < NCHUNK)
      def _():
        base = chunk(g) * CH
        pltpu.sync_copy(uv_hbm.at[:, pl.ds(base, CH)], idx.at[b])

        @pl.when(cid == 0)
        def _():
          pltpu.async_copy(nf_hbm.at[idx.at[b, 0]], stage.at[b], semg.at[b])

        @pl.when(cid == 1)
        def _():
          pltpu.async_copy(efp_hbm.at[pl.ds(base, CH)], stage.at[b],
                           semg.at[b])

    def process(g, b):
      @pl.when(chunk(g) < NCHUNK)
      def _():
        @pl.when(cid == 0)
        def _():
          pltpu.make_async_copy(nf_hbm.at[idx.at[b, 0]], stage.at[b],
                                semg.at[b]).wait()

        @pl.when(cid == 1)
        def _():
          base = chunk(g) * CH
          pltpu.make_async_copy(efp_hbm.at[pl.ds(base, CH)], stage.at[b],
                                semg.at[b]).wait()

        pltpu.sync_copy(stage.at[b], acc.at[idx.at[b, 1]], add=True)

    fetch(0, 0)

    def body(gg, _):
      for b in range(2):
        g = 2 * gg + b
        fetch(g + 1, 1 - b)
        process(g, b)
      return 0

    lax.fori_loop(0, (nit + 1) // 2, body, 0)
    plsc.subcore_barrier()

    @pl.when(cid == 0)
    def _():
      _writeout_acc(acc, sn_out, sid)

    @pl.when(cid == 1)
    def _():
      _writeout_acc(acc, sec_out, sid)

  return k(uv, nfeats, efp)


def _sc_pass_b(uvb, h1, p1, q1):
  mesh = plsc.VectorSubcoreMesh(core_axis_name="c", subcore_axis_name="s")
  nit = (NCHUNKB + NS - 1) // NS

  @functools.partial(
      pl.kernel,
      mesh=mesh,
      out_type=[
          jax.ShapeDtypeStruct((NP, H), jnp.float32),
          jax.ShapeDtypeStruct((NP, H), jnp.float32),
      ],
      scratch_types=[
          pltpu.VMEM((2, 2, CHB), jnp.int32),
          pltpu.VMEM((2, CHB, H), jnp.float32),
          pltpu.VMEM((2, CHB, H), jnp.float32),
          pltpu.VMEM_SHARED((NP, H), jnp.float32),
          pltpu.SemaphoreType.DMA((2,)),
          pltpu.SemaphoreType.DMA((2,)),
      ],
  )
  def k(uvb_hbm, h1_hbm, p1_hbm, q1_hbm, sn_out, se_out,
        idx, rows_a, rows_b, acc, sema, semb):
    cid = lax.axis_index("c")
    sid = lax.axis_index("s")

    _zero_2d(rows_a.at[0], CHB, H)
    for j in range(8):
      pltpu.sync_copy(rows_a.at[0],
                      acc.at[pl.ds(sid * ROWS_PER_TILE + j * CHB, CHB)])
    plsc.subcore_barrier()

    def chunk(g):
      return sid + g * NS

    def fetch(g, b):
      @pl.when(chunk(g) < NCHUNKB)
      def _():
        pltpu.sync_copy(uvb_hbm.at[chunk(g)], idx.at[b])

        @pl.when(cid == 0)
        def _():
          pltpu.async_copy(h1_hbm.at[idx.at[b, 0]], rows_a.at[b], sema.at[b])

        @pl.when(cid == 1)
        def _():
          pltpu.async_copy(p1_hbm.at[idx.at[b, 0]], rows_a.at[b], sema.at[b])
          pltpu.async_copy(q1_hbm.at[idx.at[b, 1]], rows_b.at[b], semb.at[b])

    def process(g, b):
      @pl.when(chunk(g) < NCHUNKB)
      def _():
        @pl.when(cid == 0)
        def _():
          pltpu.make_async_copy(h1_hbm.at[idx.at[b, 0]], rows_a.at[b],
                                sema.at[b]).wait()

        @pl.when(cid == 1)
        def _():
          pltpu.make_async_copy(p1_hbm.at[idx.at[b, 0]], rows_a.at[b],
                                sema.at[b]).wait()
          pltpu.make_async_copy(q1_hbm.at[idx.at[b, 1]], rows_b.at[b],
                                semb.at[b]).wait()

          def relu_add(r2, _):
            for dr in range(2):
              r = r2 * 2 + dr
              for kk in range(H // 16):
                s = pl.ds(kk * 16, 16)
                rows_a[b, r, s] = jnp.maximum(
                    rows_a[b, r, s] + rows_b[b, r, s], 0.0)
            return 0

          lax.fori_loop(0, CHB // 2, relu_add, 0)

        pltpu.sync_copy(rows_a.at[b], acc.at[idx.at[b, 1]], add=True)

    fetch(0, 0)

    def body(gg, _):
      for b in range(2):
        g = 2 * gg + b
        fetch(g + 1, 1 - b)
        process(g, b)
      return 0

    lax.fori_loop(0, (nit + 1) // 2, body, 0)
    plsc.subcore_barrier()

    @pl.when(cid == 0)
    def _():
      _writeout_acc(acc, sn_out, sid)

    @pl.when(cid == 1)
    def _():
      _writeout_acc(acc, se_out, sid)

  return k(uvb, h1, p1, q1)


def _sc_pass_c(uv, p2, q2):
  mesh = plsc.VectorSubcoreMesh(core_axis_name="c", subcore_axis_name="s")
  nit = (NCHUNK + NW - 1) // NW

  @functools.partial(
      pl.kernel,
      mesh=mesh,
      out_type=jax.ShapeDtypeStruct((E, H), jnp.float32),
      scratch_types=[
          pltpu.VMEM((2, 2, CH), jnp.int32),
          pltpu.VMEM((2, CH, H), jnp.float32),
          pltpu.VMEM((2, CH, H), jnp.float32),
          pltpu.SemaphoreType.DMA((2,)),
          pltpu.SemaphoreType.DMA((2,)),
      ],
  )
  def k(uv_hbm, p2_hbm, q2_hbm, e2_out,
        idx, rows_a, rows_b, sema, semb):
    cid = lax.axis_index("c")
    sid = lax.axis_index("s")
    w = sid * NC + cid

    def chunk(g):
      return w + g * NW

    def fetch(g, b):
      @pl.when(chunk(g) < NCHUNK)
      def _():
        base = chunk(g) * CH
        pltpu.sync_copy(uv_hbm.at[:, pl.ds(base, CH)], idx.at[b])
        pltpu.async_copy(p2_hbm.at[idx.at[b, 0]], rows_a.at[b], sema.at[b])
        pltpu.async_copy(q2_hbm.at[idx.at[b, 1]], rows_b.at[b], semb.at[b])

    def process(g, b):
      @pl.when(chunk(g) < NCHUNK)
      def _():
        pltpu.make_async_copy(p2_hbm.at[idx.at[b, 0]], rows_a.at[b],
                              sema.at[b]).wait()
        pltpu.make_async_copy(q2_hbm.at[idx.at[b, 1]], rows_b.at[b],
                              semb.at[b]).wait()

        def relu_add(r, _):
          for kk in range(H // 16):
            s = pl.ds(kk * 16, 16)
            rows_a[b, r, s] = jnp.maximum(rows_a[b, r, s] + rows_b[b, r, s],
                                          0.0)
          return 0

        lax.fori_loop(0, CH, relu_add, 0)
        base = chunk(g) * CH
        pltpu.sync_copy(rows_a.at[b], e2_out.at[pl.ds(base, CH)])

    fetch(0, 0)

    def body(gg, _):
      for b in range(2):
        g = 2 * gg + b
        fetch(g + 1, 1 - b)
        process(g, b)
      return 0

    lax.fori_loop(0, (nit + 1) // 2, body, 0)

  return k(uv, p2, q2)


_RB = 1024


def _tc_kernel_1(nfeats, sn, sec,
                 wan_t, wah_t, wae_t, ba, weu_t, wev_t, be):
  def body(nf, snr, secr, wan, wah, wae, b1, weu, wev, b2,
           h1o, p1o, q1o, cnto):
    sn_ = snr[...]
    se = secr[:, :DE]
    cnt = secr[:, DE:DE + 1]
    hn = jnp.where(cnt > 0, sn_ / jnp.maximum(cnt, 1.0), 0.0)
    he = jnp.where(cnt > 0, se / jnp.maximum(cnt, 1.0), 0.0)
    h = jnp.dot(nf[...], wan[...], preferred_element_type=jnp.float32)
    h += jnp.dot(hn, wah[...], preferred_element_type=jnp.float32)
    h += jnp.dot(he, wae[...], preferred_element_type=jnp.float32)
    h = jnp.maximum(h + b1[...], 0.0)
    h1o[...] = h
    p1o[...] = jnp.dot(h, weu[...], preferred_element_type=jnp.float32)
    q1o[...] = jnp.dot(h, wev[...], preferred_element_type=jnp.float32) + b2[...]
    cnto[...] = cnt + jnp.zeros((_RB, 16), jnp.float32)

  grid = (NP // _RB,)
  return pl.pallas_call(
      body,
      grid=grid,
      in_specs=[
          pl.BlockSpec((_RB, DN), lambda i: (i, 0)),
          pl.BlockSpec((_RB, DN), lambda i: (i, 0)),
          pl.BlockSpec((_RB, 128), lambda i: (i, 0)),
          pl.BlockSpec((DN, H), lambda i: (0, 0)),
          pl.BlockSpec((DN, H), lambda i: (0, 0)),
          pl.BlockSpec((DE, H), lambda i: (0, 0)),
          pl.BlockSpec((1, H), lambda i: (0, 0)),
          pl.BlockSpec((H, H), lambda i: (0, 0)),
          pl.BlockSpec((H, H), lambda i: (0, 0)),
          pl.BlockSpec((1, H), lambda i: (0, 0)),
      ],
      out_specs=[
          pl.BlockSpec((_RB, H), lambda i: (i, 0)),
          pl.BlockSpec((_RB, H), lambda i: (i, 0)),
          pl.BlockSpec((_RB, H), lambda i: (i, 0)),
          pl.BlockSpec((_RB, 16), lambda i: (i, 0)),
      ],
      out_shape=[
          jax.ShapeDtypeStruct((NP, H), jnp.float32),
          jax.ShapeDtypeStruct((NP, H), jnp.float32),
          jax.ShapeDtypeStruct((NP, H), jnp.float32),
          jax.ShapeDtypeStruct((NP, 16), jnp.float32),
      ],
  )(nfeats, sn, sec, wan_t, wah_t, wae_t, ba, weu_t, wev_t, be)


def _tc_kernel_2(h1, s2n, s2e, cnt,
                 wan_t, wah_t, wae_t, ba, weu_t, wev_t, be):
  def body(h1i, sn, se, cntp, wan, wah, wae, b1, weu, wev, b2,
           h2o, p2o, q2o):
    cnt1 = cntp[:, :1]
    hn = jnp.where(cnt1 > 0, sn[...] / jnp.maximum(cnt1, 1.0), 0.0)
    he = jnp.where(cnt1 > 0, se[...] / jnp.maximum(cnt1, 1.0), 0.0)
    h = jnp.dot(h1i[...], wan[...], preferred_element_type=jnp.float32)
    h += jnp.dot(hn, wah[...], preferred_element_type=jnp.float32)
    h += jnp.dot(he, wae[...], preferred_element_type=jnp.float32)
    h = jnp.maximum(h + b1[...], 0.0)
    h2o[...] = h
    p2o[...] = jnp.dot(h, weu[...], preferred_element_type=jnp.float32)
    q2o[...] = jnp.dot(h, wev[...], preferred_element_type=jnp.float32) + b2[...]

  grid = (NP // _RB,)
  return pl.pallas_call(
      body,
      grid=grid,
      in_specs=[
          pl.BlockSpec((_RB, H), lambda i: (i, 0)),
          pl.BlockSpec((_RB, H), lambda i: (i, 0)),
          pl.BlockSpec((_RB, H), lambda i: (i, 0)),
          pl.BlockSpec((_RB, 16), lambda i: (i, 0)),
          pl.BlockSpec((H, H), lambda i: (0, 0)),
          pl.BlockSpec((H, H), lambda i: (0, 0)),
          pl.BlockSpec((H, H), lambda i: (0, 0)),
          pl.BlockSpec((1, H), lambda i: (0, 0)),
          pl.BlockSpec((H, H), lambda i: (0, 0)),
          pl.BlockSpec((H, H), lambda i: (0, 0)),
          pl.BlockSpec((1, H), lambda i: (0, 0)),
      ],
      out_specs=[
          pl.BlockSpec((_RB, H), lambda i: (i, 0)),
          pl.BlockSpec((_RB, H), lambda i: (i, 0)),
          pl.BlockSpec((_RB, H), lambda i: (i, 0)),
      ],
      out_shape=[
          jax.ShapeDtypeStruct((NP, H), jnp.float32),
          jax.ShapeDtypeStruct((NP, H), jnp.float32),
          jax.ShapeDtypeStruct((NP, H), jnp.float32),
      ],
  )(h1, s2n, s2e, cnt, wan_t, wah_t, wae_t, ba, weu_t, wev_t, be)


def kernel(nfeats, edge_index, efeats, Wa1, ba1, We1, be1, Wa2, ba2, We2, be2):
  uv = edge_index
  nfeats_p = jnp.pad(nfeats, ((0, NP - N), (0, 0)))

  efp = _tc_kernel_0(efeats)
  sn, sec = _sc_pass_a(uv, nfeats_p, efp)
  h1, p1, q1, cnt = _tc_kernel_1(
      nfeats_p, sn, sec,
      Wa1[:, :DN].T, Wa1[:, DN:2 * DN].T, Wa1[:, 2 * DN:].T,
      ba1.reshape(1, H),
      We1[:, :H].T, We1[:, H:].T, be1.reshape(1, H))

  uvb = uv.reshape(2, NCHUNKB, CHB).transpose(1, 0, 2)
  s2n, s2e = _sc_pass_b(uvb, h1, p1, q1)
  h2, p2, q2 = _tc_kernel_2(
      h1, s2n, s2e, cnt,
      Wa2[:, :H].T, Wa2[:, H:2 * H].T, Wa2[:, 2 * H:].T,
      ba2.reshape(1, H),
      We2[:, :H].T, We2[:, H:].T, be2.reshape(1, H))

  e2 = _sc_pass_c(uv, p2, q2)
  return (h2[:N], e2)


# speedup vs baseline: 6.8009x; 1.0494x over previous
import functools

import jax
import jax.numpy as jnp
from jax import lax
from jax.experimental import pallas as pl
from jax.experimental.pallas import tpu as pltpu
from jax.experimental.pallas import tpu_sc as plsc

N = 10000
NP = 10240
E = 320000
DN = 128
DE = 16
H = 128

CH = 128
CHB = 80
NCHUNK = E // CH
NCHUNKB = E // CHB
NC = 2
NS = 16
NW = NC * NS
ROWS_PER_TILE = NP // NS


def _zero_2d(ref, nrows, ncols):
  z = jnp.zeros((16,), jnp.float32)

  def body(r, _):
    for k in range(ncols // 16):
      ref[r, pl.ds(k * 16, 16)] = z
    return 0

  lax.fori_loop(0, nrows, body, 0)


def _writeout_acc(acc, out_ref, sid):
  base_r = sid * ROWS_PER_TILE
  for j in range(5):
    off = base_r + j * 128
    pltpu.sync_copy(acc.at[pl.ds(off, 128)], out_ref.at[pl.ds(off, 128)])


_EB = 8000


def _tc_kernel_0(efeats):
  def body(ef, out):
    out[...] = jnp.concatenate(
        [ef[...],
         jnp.ones((_EB, 16), jnp.float32),
         jnp.zeros((_EB, 96), jnp.float32)], axis=1)

  return pl.pallas_call(
      body,
      grid=(E // _EB,),
      in_specs=[pl.BlockSpec((_EB, DE), lambda i: (i, 0))],
      out_specs=pl.BlockSpec((_EB, 128), lambda i: (i, 0)),
      out_shape=jax.ShapeDtypeStruct((E, 128), jnp.float32),
  )(efeats)


def _sc_pass_a(uv, nfeats, efp):
  mesh = plsc.VectorSubcoreMesh(core_axis_name="c", subcore_axis_name="s")
  nit = (NCHUNK + NS - 1) // NS

  @functools.partial(
      pl.kernel,
      mesh=mesh,
      out_type=[
          jax.ShapeDtypeStruct((NP, DN), jnp.float32),
          jax.ShapeDtypeStruct((NP, 128), jnp.float32),
      ],
      scratch_types=[
          pltpu.VMEM((2, 2, CH), jnp.int32),
          pltpu.VMEM((2, CH, 128), jnp.float32),
          pltpu.VMEM_SHARED((NP, 128), jnp.float32),
          pltpu.SemaphoreType.DMA((2,)),
      ],
  )
  def k(uv_hbm, nf_hbm, efp_hbm, sn_out, sec_out,
        idx, stage, acc, semg):
    cid = lax.axis_index("c")
    sid = lax.axis_index("s")

    _zero_2d(stage.at[0], CH, 128)
    for j in range(5):
      pltpu.sync_copy(stage.at[0],
                      acc.at[pl.ds(sid * ROWS_PER_TILE + j * 128, 128)])
    plsc.subcore_barrier()

    def chunk(g):
      return sid + g * NS

    def fetch(g, b):
      @pl.when(chunk(g) < NCHUNK)
      def _():
        base = chunk(g) * CH
        pltpu.sync_copy(uv_hbm.at[:, pl.ds(base, CH)], idx.at[b])

        @pl.when(cid == 0)
        def _():
          pltpu.async_copy(nf_hbm.at[idx.at[b, 0]], stage.at[b], semg.at[b])

        @pl.when(cid == 1)
        def _():
          pltpu.async_copy(efp_hbm.at[pl.ds(base, CH)], stage.at[b],
                           semg.at[b])

    def process(g, b):
      @pl.when(chunk(g) < NCHUNK)
      def _():
        @pl.when(cid == 0)
        def _():
          pltpu.make_async_copy(nf_hbm.at[idx.at[b, 0]], stage.at[b],
                                semg.at[b]).wait()

        @pl.when(cid == 1)
        def _():
          base = chunk(g) * CH
          pltpu.make_async_copy(efp_hbm.at[pl.ds(base, CH)], stage.at[b],
                                semg.at[b]).wait()

        pltpu.sync_copy(stage.at[b], acc.at[idx.at[b, 1]], add=True)

    fetch(0, 0)

    def body(gg, _):
      for b in range(2):
        g = 2 * gg + b
        fetch(g + 1, 1 - b)
        process(g, b)
      return 0

    lax.fori_loop(0, (nit + 1) // 2, body, 0)
    plsc.subcore_barrier()

    @pl.when(cid == 0)
    def _():
      _writeout_acc(acc, sn_out, sid)

    @pl.when(cid == 1)
    def _():
      _writeout_acc(acc, sec_out, sid)

  return k(uv, nfeats, efp)


def _sc_pass_b(uvb, h1, p1, q1):
  mesh = plsc.VectorSubcoreMesh(core_axis_name="c", subcore_axis_name="s")
  npt = NCHUNKB // NS
  nsb = npt // 5

  @functools.partial(
      pl.kernel,
      mesh=mesh,
      out_type=[
          jax.ShapeDtypeStruct((NP, H), jnp.float32),
          jax.ShapeDtypeStruct((NP, H), jnp.float32),
      ],
      scratch_types=[
          pltpu.VMEM((2, 5, 2, CHB), jnp.int32),
          pltpu.VMEM((2, CHB, H), jnp.float32),
          pltpu.VMEM((2, CHB, H), jnp.float32),
          pltpu.VMEM_SHARED((NP, H), jnp.float32),
          pltpu.SemaphoreType.DMA((2,)),
          pltpu.SemaphoreType.DMA((2,)),
      ],
  )
  def k(uvb_hbm, h1_hbm, p1_hbm, q1_hbm, sn_out, se_out,
        idx, rows_a, rows_b, acc, sema, semb):
    cid = lax.axis_index("c")
    sid = lax.axis_index("s")
    cbase = sid * npt

    _zero_2d(rows_a.at[0], CHB, H)
    for j in range(8):
      pltpu.sync_copy(rows_a.at[0],
                      acc.at[pl.ds(sid * ROWS_PER_TILE + j * CHB, CHB)])
    plsc.subcore_barrier()

    def load_sb(s, ss):
      pltpu.sync_copy(uvb_hbm.at[pl.ds(cbase + 5 * s, 5)], idx.at[ss])

    def fetch(j, ss, b):
      @pl.when(cid == 0)
      def _():
        pltpu.async_copy(h1_hbm.at[idx.at[ss, j, 0]], rows_a.at[b],
                         sema.at[b])

      @pl.when(cid == 1)
      def _():
        pltpu.async_copy(p1_hbm.at[idx.at[ss, j, 0]], rows_a.at[b],
                         sema.at[b])
        pltpu.async_copy(q1_hbm.at[idx.at[ss, j, 1]], rows_b.at[b],
                         semb.at[b])

    def process(j, ss, b):
      @pl.when(cid == 0)
      def _():
        pltpu.make_async_copy(h1_hbm.at[idx.at[ss, j, 0]], rows_a.at[b],
                              sema.at[b]).wait()

      @pl.when(cid == 1)
      def _():
        pltpu.make_async_copy(p1_hbm.at[idx.at[ss, j, 0]], rows_a.at[b],
                              sema.at[b]).wait()
        pltpu.make_async_copy(q1_hbm.at[idx.at[ss, j, 1]], rows_b.at[b],
                              semb.at[b]).wait()

        def relu_add(r2, _):
          for dr in range(2):
            r = r2 * 2 + dr
            for kk in range(H // 16):
              sdd = pl.ds(kk * 16, 16)
              rows_a[b, r, sdd] = jnp.maximum(
                  rows_a[b, r, sdd] + rows_b[b, r, sdd], 0.0)
          return 0

        lax.fori_loop(0, CHB // 2, relu_add, 0)

      pltpu.sync_copy(rows_a.at[b], acc.at[idx.at[ss, j, 1]], add=True)

    load_sb(0, 0)
    fetch(0, 0, 0)

    def body(gg, _):
      for ss in range(2):
        s = 2 * gg + ss
        for j in range(5):
          b = (ss + j) % 2
          b1 = 1 - b
          if j == 4:
            @pl.when(s < nsb - 1)
            def _(s=s, ss=ss, b1=b1):
              load_sb(s + 1, 1 - ss)
              fetch(0, 1 - ss, b1)
          else:
            fetch(j + 1, ss, b1)
          process(j, ss, b)
      return 0

    lax.fori_loop(0, nsb // 2, body, 0)
    plsc.subcore_barrier()

    @pl.when(cid == 0)
    def _():
      _writeout_acc(acc, sn_out, sid)

    @pl.when(cid == 1)
    def _():
      _writeout_acc(acc, se_out, sid)

  return k(uvb, h1, p1, q1)


def _sc_pass_c(uv, p2, q2):
  mesh = plsc.VectorSubcoreMesh(core_axis_name="c", subcore_axis_name="s")
  nit = (NCHUNK + NW - 1) // NW

  @functools.partial(
      pl.kernel,
      mesh=mesh,
      out_type=jax.ShapeDtypeStruct((E, H), jnp.float32),
      scratch_types=[
          pltpu.VMEM((2, 2, CH), jnp.int32),
          pltpu.VMEM((2, CH, H), jnp.float32),
          pltpu.VMEM((2, CH, H), jnp.float32),
          pltpu.SemaphoreType.DMA((2,)),
          pltpu.SemaphoreType.DMA((2,)),
      ],
  )
  def k(uv_hbm, p2_hbm, q2_hbm, e2_out,
        idx, rows_a, rows_b, sema, semb):
    cid = lax.axis_index("c")
    sid = lax.axis_index("s")
    w = sid * NC + cid

    def chunk(g):
      return w + g * NW

    def fetch(g, b):
      @pl.when(chunk(g) < NCHUNK)
      def _():
        base = chunk(g) * CH
        pltpu.sync_copy(uv_hbm.at[:, pl.ds(base, CH)], idx.at[b])
        pltpu.async_copy(p2_hbm.at[idx.at[b, 0]], rows_a.at[b], sema.at[b])
        pltpu.async_copy(q2_hbm.at[idx.at[b, 1]], rows_b.at[b], semb.at[b])

    def process(g, b):
      @pl.when(chunk(g) < NCHUNK)
      def _():
        pltpu.make_async_copy(p2_hbm.at[idx.at[b, 0]], rows_a.at[b],
                              sema.at[b]).wait()
        pltpu.make_async_copy(q2_hbm.at[idx.at[b, 1]], rows_b.at[b],
                              semb.at[b]).wait()

        def relu_add(r, _):
          for kk in range(H // 16):
            s = pl.ds(kk * 16, 16)
            rows_a[b, r, s] = jnp.maximum(rows_a[b, r, s] + rows_b[b, r, s],
                                          0.0)
          return 0

        lax.fori_loop(0, CH, relu_add, 0)
        base = chunk(g) * CH
        pltpu.sync_copy(rows_a.at[b], e2_out.at[pl.ds(base, CH)])

    fetch(0, 0)

    def body(gg, _):
      for b in range(2):
        g = 2 * gg + b
        fetch(g + 1, 1 - b)
        process(g, b)
      return 0

    lax.fori_loop(0, (nit + 1) // 2, body, 0)

  return k(uv, p2, q2)


_RB = 1024


def _tc_kernel_1(nfeats, sn, sec,
                 wan_t, wah_t, wae_t, ba, weu_t, wev_t, be):
  def body(nf, snr, secr, wan, wah, wae, b1, weu, wev, b2,
           h1o, p1o, q1o, cnto):
    sn_ = snr[...]
    se = secr[:, :DE]
    cnt = secr[:, DE:DE + 1]
    hn = jnp.where(cnt > 0, sn_ / jnp.maximum(cnt, 1.0), 0.0)
    he = jnp.where(cnt > 0, se / jnp.maximum(cnt, 1.0), 0.0)
    h = jnp.dot(nf[...], wan[...], preferred_element_type=jnp.float32)
    h += jnp.dot(hn, wah[...], preferred_element_type=jnp.float32)
    h += jnp.dot(he, wae[...], preferred_element_type=jnp.float32)
    h = jnp.maximum(h + b1[...], 0.0)
    h1o[...] = h
    p1o[...] = jnp.dot(h, weu[...], preferred_element_type=jnp.float32)
    q1o[...] = jnp.dot(h, wev[...], preferred_element_type=jnp.float32) + b2[...]
    cnto[...] = cnt + jnp.zeros((_RB, 16), jnp.float32)

  grid = (NP // _RB,)
  return pl.pallas_call(
      body,
      grid=grid,
      in_specs=[
          pl.BlockSpec((_RB, DN), lambda i: (i, 0)),
          pl.BlockSpec((_RB, DN), lambda i: (i, 0)),
          pl.BlockSpec((_RB, 128), lambda i: (i, 0)),
          pl.BlockSpec((DN, H), lambda i: (0, 0)),
          pl.BlockSpec((DN, H), lambda i: (0, 0)),
          pl.BlockSpec((DE, H), lambda i: (0, 0)),
          pl.BlockSpec((1, H), lambda i: (0, 0)),
          pl.BlockSpec((H, H), lambda i: (0, 0)),
          pl.BlockSpec((H, H), lambda i: (0, 0)),
          pl.BlockSpec((1, H), lambda i: (0, 0)),
      ],
      out_specs=[
          pl.BlockSpec((_RB, H), lambda i: (i, 0)),
          pl.BlockSpec((_RB, H), lambda i: (i, 0)),
          pl.BlockSpec((_RB, H), lambda i: (i, 0)),
          pl.BlockSpec((_RB, 16), lambda i: (i, 0)),
      ],
      out_shape=[
          jax.ShapeDtypeStruct((NP, H), jnp.float32),
          jax.ShapeDtypeStruct((NP, H), jnp.float32),
          jax.ShapeDtypeStruct((NP, H), jnp.float32),
          jax.ShapeDtypeStruct((NP, 16), jnp.float32),
      ],
  )(nfeats, sn, sec, wan_t, wah_t, wae_t, ba, weu_t, wev_t, be)


def _tc_kernel_2(h1, s2n, s2e, cnt,
                 wan_t, wah_t, wae_t, ba, weu_t, wev_t, be):
  def body(h1i, sn, se, cntp, wan, wah, wae, b1, weu, wev, b2,
           h2o, p2o, q2o):
    cnt1 = cntp[:, :1]
    hn = jnp.where(cnt1 > 0, sn[...] / jnp.maximum(cnt1, 1.0), 0.0)
    he = jnp.where(cnt1 > 0, se[...] / jnp.maximum(cnt1, 1.0), 0.0)
    h = jnp.dot(h1i[...], wan[...], preferred_element_type=jnp.float32)
    h += jnp.dot(hn, wah[...], preferred_element_type=jnp.float32)
    h += jnp.dot(he, wae[...], preferred_element_type=jnp.float32)
    h = jnp.maximum(h + b1[...], 0.0)
    h2o[...] = h
    p2o[...] = jnp.dot(h, weu[...], preferred_element_type=jnp.float32)
    q2o[...] = jnp.dot(h, wev[...], preferred_element_type=jnp.float32) + b2[...]

  grid = (NP // _RB,)
  return pl.pallas_call(
      body,
      grid=grid,
      in_specs=[
          pl.BlockSpec((_RB, H), lambda i: (i, 0)),
          pl.BlockSpec((_RB, H), lambda i: (i, 0)),
          pl.BlockSpec((_RB, H), lambda i: (i, 0)),
          pl.BlockSpec((_RB, 16), lambda i: (i, 0)),
          pl.BlockSpec((H, H), lambda i: (0, 0)),
          pl.BlockSpec((H, H), lambda i: (0, 0)),
          pl.BlockSpec((H, H), lambda i: (0, 0)),
          pl.BlockSpec((1, H), lambda i: (0, 0)),
          pl.BlockSpec((H, H), lambda i: (0, 0)),
          pl.BlockSpec((H, H), lambda i: (0, 0)),
          pl.BlockSpec((1, H), lambda i: (0, 0)),
      ],
      out_specs=[
          pl.BlockSpec((_RB, H), lambda i: (i, 0)),
          pl.BlockSpec((_RB, H), lambda i: (i, 0)),
          pl.BlockSpec((_RB, H), lambda i: (i, 0)),
      ],
      out_shape=[
          jax.ShapeDtypeStruct((NP, H), jnp.float32),
          jax.ShapeDtypeStruct((NP, H), jnp.float32),
          jax.ShapeDtypeStruct((NP, H), jnp.float32),
      ],
  )(h1, s2n, s2e, cnt, wan_t, wah_t, wae_t, ba, weu_t, wev_t, be)


def kernel(nfeats, edge_index, efeats, Wa1, ba1, We1, be1, Wa2, ba2, We2, be2):
  uv = edge_index
  nfeats_p = jnp.pad(nfeats, ((0, NP - N), (0, 0)))

  efp = _tc_kernel_0(efeats)
  sn, sec = _sc_pass_a(uv, nfeats_p, efp)
  h1, p1, q1, cnt = _tc_kernel_1(
      nfeats_p, sn, sec,
      Wa1[:, :DN].T, Wa1[:, DN:2 * DN].T, Wa1[:, 2 * DN:].T,
      ba1.reshape(1, H),
      We1[:, :H].T, We1[:, H:].T, be1.reshape(1, H))

  uvb = uv.reshape(2, NCHUNKB, CHB).transpose(1, 0, 2)
  s2n, s2e = _sc_pass_b(uvb, h1, p1, q1)
  h2, p2, q2 = _tc_kernel_2(
      h1, s2n, s2e, cnt,
      Wa2[:, :H].T, Wa2[:, H:2 * H].T, Wa2[:, 2 * H:].T,
      ba2.reshape(1, H),
      We2[:, :H].T, We2[:, H:].T, be2.reshape(1, H))

  e2 = _sc_pass_c(uv, p2, q2)
  return (h2[:N], e2)
